# Initial kernel scaffold; baseline (speedup 1.0000x reference)
import jax
import jax.numpy as jnp
from jax.experimental import pallas as pl


def kernel(node_fts, edge_fts, node_hints, edge_hints, W_node, b_node, W_edge, b_edge, batch_vec, edge_index, processor_step, training_step, teacher_force):
    raise NotImplementedError("write your pallas kernel here")


# speedup vs baseline: 1.0462x; 1.0462x over previous
import functools

import jax
import jax.numpy as jnp
from jax import lax
from jax.experimental import pallas as pl

N_NODES = 10000
N_EDGES = 320000
H = 128
NUM_GRAPHS = 128
EBLK = 12800


def _edge_logits_body(fts_ref, w_ref, b_ref, out_ref):
    x = fts_ref[...]
    w = w_ref[...]
    out = lax.dot_general(w, x, (((1,), (1,)), ((), ())))
    out_ref[...] = out + b_ref[...]


def _node_logits_body(fts_ref, w_ref, b_ref, out_ref):
    x = fts_ref[...]
    w = w_ref[...]
    out = lax.dot_general(w, x, (((1,), (1,)), ((), ())))
    out_ref[...] = out + b_ref[...]


def _edge_logits(edge_fts, W_edge, b_edge):
    return pl.pallas_call(
        _edge_logits_body,
        grid=(N_EDGES // EBLK,),
        in_specs=[
            pl.BlockSpec((EBLK, H), lambda i: (i, 0)),
            pl.BlockSpec((2, H), lambda i: (0, 0)),
            pl.BlockSpec((2, 1), lambda i: (0, 0)),
        ],
        out_specs=pl.BlockSpec((2, EBLK), lambda i: (0, i)),
        out_shape=jax.ShapeDtypeStruct((2, N_EDGES), jnp.float32),
    )(edge_fts, W_edge, b_edge.reshape(2, 1))


def _node_logits(node_fts, W_node, b_node):
    return pl.pallas_call(
        _node_logits_body,
        grid=(1,),
        in_specs=[
            pl.BlockSpec((N_NODES, H), lambda i: (0, 0)),
            pl.BlockSpec((3, H), lambda i: (0, 0)),
            pl.BlockSpec((3, 1), lambda i: (0, 0)),
        ],
        out_specs=pl.BlockSpec((3, N_NODES), lambda i: (0, 0)),
        out_shape=jax.ShapeDtypeStruct((3, N_NODES), jnp.float32),
    )(node_fts, W_node, b_node.reshape(3, 1))


def _seg_log_softmax(logits, index, num_segments):
    m = jax.ops.segment_max(logits, index, num_segments=num_segments)
    m = jnp.where(jnp.isfinite(m), m, 0.0)
    shifted = logits - m[index]
    denom = jax.ops.segment_sum(jnp.exp(shifted), index, num_segments=num_segments)
    return shifted - jnp.log(denom + 1e-20)[index], m


def kernel(node_fts, edge_fts, node_hints, edge_hints, W_node, b_node, W_edge,
           b_edge, batch_vec, edge_index, processor_step, training_step,
           teacher_force):
    logits_n = _node_logits(node_fts, W_node, b_node)
    logits_e = _edge_logits(edge_fts, W_edge, b_edge)

    gt_n = lax.dynamic_slice_in_dim(node_hints, processor_step, 1, axis=1)
    gt_n = gt_n[:, 0, :]
    gt_e = lax.dynamic_slice_in_dim(edge_hints, processor_step, 1, axis=1)
    gt_e = gt_e[:, 0, :]

    loss = 0.0
    l_n0 = logits_n[0]
    g_n0 = gt_n[:, 0]
    logp, m_n = _seg_log_softmax(l_n0, batch_vec, NUM_GRAPHS)
    loss = loss - jnp.sum(g_n0 * logp) / NUM_GRAPHS
    pred_n0 = (l_n0 >= m_n[batch_vec]).astype(jnp.float32)
    preds_n = [pred_n0]
    for idx in (1, 2):
        l = logits_n[idx]
        g = gt_n[:, idx]
        loss = loss + jnp.mean(jnp.maximum(l, 0.0) - l * g +
                               jnp.log1p(jnp.exp(-jnp.abs(l))))
        preds_n.append((l > 0.0).astype(jnp.float32))

    e_idx = edge_index[0]
    l_e0 = logits_e[0]
    g_e0 = gt_e[:, 0]
    logp, m_e0 = _seg_log_softmax(l_e0, e_idx, N_NODES)
    loss = loss - jnp.sum(g_e0 * logp) / N_NODES
    pred_e0 = (l_e0 >= m_e0[e_idx]).astype(jnp.float32)
    idx2 = batch_vec[e_idx]
    l_e1 = logits_e[1]
    g_e1 = gt_e[:, 1]
    weight = jnp.sum(batch_vec == 0).astype(jnp.float32)
    logp, m_e1 = _seg_log_softmax(l_e1, idx2, NUM_GRAPHS)
    loss = loss - weight * jnp.sum(g_e1 * logp) / NUM_GRAPHS
    pred_e1 = (l_e1 >= m_e1[idx2]).astype(jnp.float32)

    preds_e = [pred_e0, pred_e1]
    states_n = jnp.stack(preds_n, axis=-1)
    states_e = jnp.stack(preds_e, axis=-1)
    states_n = jnp.where(teacher_force, gt_n, states_n)
    states_e = jnp.where(teacher_force, gt_e, states_e)
    return (states_n, states_e, loss)


# speedup vs baseline: 15.2288x; 14.5561x over previous
import functools

import jax
import jax.numpy as jnp
from jax import lax
from jax.experimental import pallas as pl
from jax.experimental.pallas import tpu as pltpu
from jax.experimental.pallas import tpu_sc as plsc

N_NODES = 10000
N_EDGES = 320000
H = 128
G = 128
EBLK = 12800
S0P = 10112
NW = 32
CH = N_EDGES // NW
L = 16
NEG = -3.4e38

_SC_PARAMS = pltpu.CompilerParams(needs_layout_passes=False)


def _sc_mesh():
    return plsc.VectorSubcoreMesh(
        core_axis_name="c", subcore_axis_name="s", num_cores=2, num_subcores=16)


def _a_body(fts_ref, w_ref, b_ref, gt_ref, out_ref, dots_ref):
    i = pl.program_id(0)
    lg = lax.dot_general(w_ref[...], fts_ref[...],
                         (((1,), (1,)), ((), ())))
    lg = lg + b_ref[...]
    out_ref[...] = lg
    d = lax.dot_general(lg, gt_ref[...], (((1,), (0,)), ((), ())))

    @pl.when(i == 0)
    def _():
        dots_ref[...] = jnp.zeros_like(dots_ref)

    dots_ref[...] += d


def _edge_logits(edge_fts, W_edge, b_edge, gt_e):
    return pl.pallas_call(
        _a_body,
        grid=(N_EDGES // EBLK,),
        in_specs=[
            pl.BlockSpec((EBLK, H), lambda i: (i, 0)),
            pl.BlockSpec((2, H), lambda i: (0, 0)),
            pl.BlockSpec((2, 1), lambda i: (0, 0)),
            pl.BlockSpec((EBLK, 2), lambda i: (i, 0)),
        ],
        out_specs=[
            pl.BlockSpec((2, EBLK), lambda i: (0, i)),
            pl.BlockSpec((2, 2), lambda i: (0, 0)),
        ],
        out_shape=[
            jax.ShapeDtypeStruct((2, N_EDGES), jnp.float32),
            jax.ShapeDtypeStruct((2, 2), jnp.float32),
        ],
    )(edge_fts, W_edge, b_edge.reshape(2, 1), gt_e)


def _scatter_max16(acc, idx, val):
    def cond(act):
        return jnp.any(act)

    def body(act):
        cur = plsc.load_gather(acc, [idx])
        need = jnp.logical_and(act, val > cur)
        plsc.store_scatter(acc, [idx], val, mask=need)
        cur2 = plsc.load_gather(acc, [idx])
        return jnp.logical_and(need, val > cur2)

    act0 = val > plsc.load_gather(acc, [idx])
    lax.while_loop(cond, body, act0)


def _vfill(ref, n, value, dtype):
    def body(i, _):
        ref[pl.ds(i * L, L)] = jnp.full((L,), value, dtype)
        return 0

    lax.fori_loop(0, n // L, body, 0)


def _k1_partials(e_idx, l0, g0, l1, g1, batch_vec):
    @functools.partial(
        pl.kernel,
        out_type=(
            jax.ShapeDtypeStruct((NW, S0P), jnp.float32),
            jax.ShapeDtypeStruct((NW, S0P), jnp.float32),
            jax.ShapeDtypeStruct((NW, G), jnp.float32),
            jax.ShapeDtypeStruct((NW, G), jnp.float32),
        ),
        mesh=_sc_mesh(),
        compiler_params=_SC_PARAMS,
        scratch_types=[
            pltpu.VMEM((CH,), jnp.int32),
            pltpu.VMEM((CH,), jnp.float32),
            pltpu.VMEM((CH,), jnp.float32),
            pltpu.VMEM((CH,), jnp.float32),
            pltpu.VMEM((CH,), jnp.float32),
            pltpu.VMEM((N_NODES,), jnp.int32),
            pltpu.VMEM((S0P,), jnp.float32),
            pltpu.VMEM((S0P,), jnp.float32),
            pltpu.VMEM((G,), jnp.float32),
            pltpu.VMEM((G,), jnp.float32),
        ],
    )
    def k(idx_h, l0_h, g0_h, l1_h, g1_h, bv_h, m0p_h, g0p_h, m2p_h, g2p_h,
          idx_v, l0_v, g0_v, l1_v, g1_v, bv_v, m0a, g0a, m2a, g2a):
        wid = lax.axis_index("s") * 2 + lax.axis_index("c")
        base = wid * CH
        pltpu.sync_copy(idx_h.at[pl.ds(base, CH)], idx_v)
        pltpu.sync_copy(l0_h.at[pl.ds(base, CH)], l0_v)
        pltpu.sync_copy(g0_h.at[pl.ds(base, CH)], g0_v)
        pltpu.sync_copy(l1_h.at[pl.ds(base, CH)], l1_v)
        pltpu.sync_copy(g1_h.at[pl.ds(base, CH)], g1_v)
        pltpu.sync_copy(bv_h, bv_v)
        _vfill(m0a, S0P, NEG, jnp.float32)
        _vfill(g0a, S0P, 0.0, jnp.float32)
        _vfill(m2a, G, NEG, jnp.float32)
        _vfill(g2a, G, 0.0, jnp.float32)

        def step(j, _):
            sl = pl.ds(j * L, L)
            idx = idx_v[sl]
            _scatter_max16(m0a, idx, l0_v[sl])
            plsc.addupdate_scatter(g0a, [idx], g0_v[sl])
            idx2 = plsc.load_gather(bv_v, [idx])
            _scatter_max16(m2a, idx2, l1_v[sl])
            plsc.addupdate_scatter(g2a, [idx2], g1_v[sl])
            return 0

        lax.fori_loop(0, CH // L, step, 0)
        pltpu.sync_copy(m0a, m0p_h.at[wid])
        pltpu.sync_copy(g0a, g0p_h.at[wid])
        pltpu.sync_copy(m2a, m2p_h.at[wid])
        pltpu.sync_copy(g2a, g2p_h.at[wid])

    return k(e_idx, l0, g0, l1, g1, batch_vec)


def _c1_body(m0p_ref, g0p_ref, m2p_ref, g2p_ref, nf_ref, w_ref, b_ref,
             gt_ref, bv_ref, m0f_ref, g0f_ref, m2f_ref, g2f_ref,
             states_ref, np_ref):
    m0f_ref[...] = jnp.max(m0p_ref[...], axis=0, keepdims=True)
    g0f_ref[...] = jnp.sum(g0p_ref[...], axis=0, keepdims=True)
    m2f_ref[...] = jnp.max(m2p_ref[...], axis=0, keepdims=True)
    g2f_ref[...] = jnp.sum(g2p_ref[...], axis=0, keepdims=True)

    logits = lax.dot_general(nf_ref[...], w_ref[...],
                             (((1,), (1,)), ((), ())))
    logits = logits + b_ref[...]
    gt = gt_ref[...]
    bv = bv_ref[...]
    onehot = bv == lax.broadcasted_iota(jnp.int32, (N_NODES, G), 1)

    l0 = logits[:, 0:1]
    g0 = gt[:, 0:1]
    m_n = jnp.max(jnp.where(onehot, l0, NEG), axis=0, keepdims=True)
    gseg = jnp.sum(jnp.where(onehot, g0, 0.0), axis=0, keepdims=True)
    m_gath = jnp.sum(jnp.where(onehot, m_n, 0.0), axis=1, keepdims=True)
    denom = jnp.sum(jnp.where(onehot, jnp.exp(l0 - m_gath), 0.0),
                    axis=0, keepdims=True)
    loss_n0 = (-jnp.sum(g0 * l0) + jnp.sum(gseg * m_n) +
               jnp.sum(gseg * jnp.log(denom + 1e-20))) / G
    pred0 = (l0 >= m_gath).astype(jnp.float32)

    l1 = logits[:, 1:2]
    g1 = gt[:, 1:2]
    bce1 = jnp.sum(jnp.maximum(l1, 0.0) - l1 * g1 +
                   jnp.log1p(jnp.exp(-jnp.abs(l1)))) / N_NODES
    l2 = logits[:, 2:3]
    g2 = gt[:, 2:3]
    bce2 = jnp.sum(jnp.maximum(l2, 0.0) - l2 * g2 +
                   jnp.log1p(jnp.exp(-jnp.abs(l2)))) / N_NODES
    states_ref[...] = jnp.concatenate(
        [pred0, (l1 > 0.0).astype(jnp.float32),
         (l2 > 0.0).astype(jnp.float32)], axis=1)

    n0count = jnp.sum(jnp.where(onehot[:, 0:1], 1.0, 0.0))
    loss_node = loss_n0 + bce1 + bce2
    ii = lax.broadcasted_iota(jnp.int32, (1, G), 1)
    np_ref[...] = jnp.where(ii == 0, loss_node,
                            jnp.where(ii == 1, n0count, 0.0))


def _c1(m0p, g0p, m2p, g2p, node_fts, W_node, b_node, gt_n, batch_vec):
    return pl.pallas_call(
        _c1_body,
        grid=(1,),
        in_specs=[
            pl.BlockSpec((NW, S0P), lambda i: (0, 0)),
            pl.BlockSpec((NW, S0P), lambda i: (0, 0)),
            pl.BlockSpec((NW, G), lambda i: (0, 0)),
            pl.BlockSpec((NW, G), lambda i: (0, 0)),
            pl.BlockSpec((N_NODES, H), lambda i: (0, 0)),
            pl.BlockSpec((3, H), lambda i: (0, 0)),
            pl.BlockSpec((1, 3), lambda i: (0, 0)),
            pl.BlockSpec((N_NODES, 3), lambda i: (0, 0)),
            pl.BlockSpec((N_NODES, 1), lambda i: (0, 0)),
        ],
        out_specs=[
            pl.BlockSpec((1, S0P), lambda i: (0, 0)),
            pl.BlockSpec((1, S0P), lambda i: (0, 0)),
            pl.BlockSpec((1, G), lambda i: (0, 0)),
            pl.BlockSpec((1, G), lambda i: (0, 0)),
            pl.BlockSpec((N_NODES, 3), lambda i: (0, 0)),
            pl.BlockSpec((1, G), lambda i: (0, 0)),
        ],
        out_shape=[
            jax.ShapeDtypeStruct((1, S0P), jnp.float32),
            jax.ShapeDtypeStruct((1, S0P), jnp.float32),
            jax.ShapeDtypeStruct((1, G), jnp.float32),
            jax.ShapeDtypeStruct((1, G), jnp.float32),
            jax.ShapeDtypeStruct((N_NODES, 3), jnp.float32),
            jax.ShapeDtypeStruct((1, G), jnp.float32),
        ],
    )(m0p, g0p, m2p, g2p, node_fts, W_node, b_node.reshape(1, 3), gt_n,
      batch_vec.reshape(N_NODES, 1))


def _k3_denoms_preds(e_idx, l0, l1, batch_vec, m0f, m2f):
    @functools.partial(
        pl.kernel,
        out_type=(
            jax.ShapeDtypeStruct((N_EDGES,), jnp.float32),
            jax.ShapeDtypeStruct((N_EDGES,), jnp.float32),
            jax.ShapeDtypeStruct((NW, S0P), jnp.float32),
            jax.ShapeDtypeStruct((NW, G), jnp.float32),
        ),
        mesh=_sc_mesh(),
        compiler_params=_SC_PARAMS,
        scratch_types=[
            pltpu.VMEM((CH,), jnp.int32),
            pltpu.VMEM((CH,), jnp.float32),
            pltpu.VMEM((CH,), jnp.float32),
            pltpu.VMEM((N_NODES,), jnp.int32),
            pltpu.VMEM((S0P,), jnp.float32),
            pltpu.VMEM((G,), jnp.float32),
            pltpu.VMEM((CH,), jnp.float32),
            pltpu.VMEM((CH,), jnp.float32),
            pltpu.VMEM((S0P,), jnp.float32),
            pltpu.VMEM((G,), jnp.float32),
        ],
    )
    def k(idx_h, l0_h, l1_h, bv_h, m0f_h, m2f_h, p0_h, p1_h, d0p_h, d2p_h,
          idx_v, l0_v, l1_v, bv_v, m0f_v, m2f_v, p0_v, p1_v, d0a, d2a):
        wid = lax.axis_index("s") * 2 + lax.axis_index("c")
        base = wid * CH
        pltpu.sync_copy(idx_h.at[pl.ds(base, CH)], idx_v)
        pltpu.sync_copy(l0_h.at[pl.ds(base, CH)], l0_v)
        pltpu.sync_copy(l1_h.at[pl.ds(base, CH)], l1_v)
        pltpu.sync_copy(bv_h, bv_v)
        pltpu.sync_copy(m0f_h, m0f_v)
        pltpu.sync_copy(m2f_h, m2f_v)
        _vfill(d0a, S0P, 0.0, jnp.float32)
        _vfill(d2a, G, 0.0, jnp.float32)

        def step(j, _):
            sl = pl.ds(j * L, L)
            idx = idx_v[sl]
            v0 = l0_v[sl]
            m0g = plsc.load_gather(m0f_v, [idx])
            p0_v[sl] = (v0 >= m0g).astype(jnp.float32)
            plsc.addupdate_scatter(d0a, [idx], jnp.exp(v0 - m0g))
            idx2 = plsc.load_gather(bv_v, [idx])
            v1 = l1_v[sl]
            m2g = plsc.load_gather(m2f_v, [idx2])
            p1_v[sl] = (v1 >= m2g).astype(jnp.float32)
            plsc.addupdate_scatter(d2a, [idx2], jnp.exp(v1 - m2g))
            return 0

        lax.fori_loop(0, CH // L, step, 0)
        pltpu.sync_copy(p0_v, p0_h.at[pl.ds(base, CH)])
        pltpu.sync_copy(p1_v, p1_h.at[pl.ds(base, CH)])
        pltpu.sync_copy(d0a, d0p_h.at[wid])
        pltpu.sync_copy(d2a, d2p_h.at[wid])

    return k(e_idx, l0, l1, batch_vec, m0f, m2f)


def _c2_body(d0p_ref, d2p_ref, m0f_ref, g0f_ref, m2f_ref, g2f_ref,
             dots_ref, np_ref, out_ref):
    d0f = jnp.sum(d0p_ref[...], axis=0, keepdims=True)
    d2f = jnp.sum(d2p_ref[...], axis=0, keepdims=True)
    dots = dots_ref[...]
    dot0 = dots[0, 0]
    dot1 = dots[1, 1]
    npv = np_ref[...]
    ii = lax.broadcasted_iota(jnp.int32, (1, G), 1)
    loss_node = jnp.sum(jnp.where(ii == 0, npv, 0.0))
    n0count = jnp.sum(jnp.where(ii == 1, npv, 0.0))
    g0f = g0f_ref[...]
    m0f = m0f_ref[...]
    g2f = g2f_ref[...]
    m2f = m2f_ref[...]
    loss_a = (-dot0 + jnp.sum(g0f * m0f) +
              jnp.sum(g0f * jnp.log(d0f + 1e-20))) / N_NODES
    loss_b = n0count * (-dot1 + jnp.sum(g2f * m2f) +
                        jnp.sum(g2f * jnp.log(d2f + 1e-20))) / G
    out_ref[...] = jnp.full((1, 1), loss_node + loss_a + loss_b, jnp.float32)


def _c2(d0p, d2p, m0f, g0f, m2f, g2f, dots, npart):
    return pl.pallas_call(
        _c2_body,
        grid=(1,),
        in_specs=[
            pl.BlockSpec((NW, S0P), lambda i: (0, 0)),
            pl.BlockSpec((NW, G), lambda i: (0, 0)),
            pl.BlockSpec((1, S0P), lambda i: (0, 0)),
            pl.BlockSpec((1, S0P), lambda i: (0, 0)),
            pl.BlockSpec((1, G), lambda i: (0, 0)),
            pl.BlockSpec((1, G), lambda i: (0, 0)),
            pl.BlockSpec((2, 2), lambda i: (0, 0)),
            pl.BlockSpec((1, G), lambda i: (0, 0)),
        ],
        out_specs=pl.BlockSpec((1, 1), lambda i: (0, 0)),
        out_shape=jax.ShapeDtypeStruct((1, 1), jnp.float32),
    )(d0p, d2p, m0f, g0f, m2f, g2f, dots, npart)


def kernel(node_fts, edge_fts, node_hints, edge_hints, W_node, b_node, W_edge,
           b_edge, batch_vec, edge_index, processor_step, training_step,
           teacher_force):
    gt_n = lax.dynamic_slice_in_dim(node_hints, processor_step, 1, axis=1)
    gt_n = gt_n[:, 0, :]
    gt_e = lax.dynamic_slice_in_dim(edge_hints, processor_step, 1, axis=1)
    gt_e = gt_e[:, 0, :]
    e_idx = edge_index[0]
    batch_vec = batch_vec.astype(jnp.int32)
    e_idx = e_idx.astype(jnp.int32)

    logits_eT, dots = _edge_logits(edge_fts, W_edge, b_edge, gt_e)
    l0 = logits_eT[0]
    l1 = logits_eT[1]
    g0 = gt_e[:, 0]
    g1 = gt_e[:, 1]

    m0p, g0p, m2p, g2p = _k1_partials(e_idx, l0, g0, l1, g1, batch_vec)
    m0f, g0f, m2f, g2f, states_n, npart = _c1(
        m0p, g0p, m2p, g2p, node_fts, W_node, b_node, gt_n, batch_vec)
    pred0, pred1, d0p, d2p = _k3_denoms_preds(
        e_idx, l0, l1, batch_vec, m0f.reshape(S0P), m2f.reshape(G))
    loss11 = _c2(d0p, d2p, m0f, g0f, m2f, g2f, dots, npart)

    loss = loss11[0, 0]
    states_e = jnp.stack([pred0, pred1], axis=-1)
    states_n = jnp.where(teacher_force, gt_n, states_n)
    states_e = jnp.where(teacher_force, gt_e, states_e)
    return (states_n, states_e, loss)


# speedup vs baseline: 17.0083x; 1.1168x over previous
import functools

import jax
import jax.numpy as jnp
from jax import lax
from jax.experimental import pallas as pl
from jax.experimental.pallas import tpu as pltpu
from jax.experimental.pallas import tpu_sc as plsc

N_NODES = 10000
N_EDGES = 320000
H = 128
G = 128
EBLK = 12800
S0P = 10112
NW = 32
CH = N_EDGES // NW
L = 16
NEG = -3.4e38

_SC_PARAMS = pltpu.CompilerParams(needs_layout_passes=False)


def _sc_mesh():
    return plsc.VectorSubcoreMesh(
        core_axis_name="c", subcore_axis_name="s", num_cores=2, num_subcores=16)


def _a_body(step_ref, fts_ref, w_ref, b_ref, h32_ref, l0_ref, l1_ref,
            g0_ref, g1_ref, dots_ref):
    i = pl.program_id(0)
    lg = lax.dot_general(w_ref[...], fts_ref[...],
                         (((1,), (1,)), ((), ())))
    lg = lg + b_ref[...]
    step = step_ref[0]
    bi = lax.broadcasted_iota(jnp.int32, (32, 2), 0)
    bj = lax.broadcasted_iota(jnp.int32, (32, 2), 1)
    oh = (bi == 2 * step + bj).astype(jnp.float32)
    gtT = lax.dot_general(oh, h32_ref[...],
                          (((0,), (1,)), ((), ())))
    l0_ref[...] = lg[0:1, :]
    l1_ref[...] = lg[1:2, :]
    g0_ref[...] = gtT[0:1, :]
    g1_ref[...] = gtT[1:2, :]
    d = lax.dot_general(lg, gtT, (((1,), (1,)), ((), ())))

    @pl.when(i == 0)
    def _():
        dots_ref[...] = jnp.zeros_like(dots_ref)

    dots_ref[...] += d


def _edge_logits(edge_fts, W_edge, b_edge, h32, step_i):
    n = N_EDGES
    return pl.pallas_call(
        _a_body,
        grid=(N_EDGES // EBLK,),
        in_specs=[
            pl.BlockSpec(memory_space=pltpu.SMEM),
            pl.BlockSpec((EBLK, H), lambda i: (i, 0)),
            pl.BlockSpec((2, H), lambda i: (0, 0)),
            pl.BlockSpec((2, 1), lambda i: (0, 0)),
            pl.BlockSpec((EBLK, 32), lambda i: (i, 0)),
        ],
        out_specs=[
            pl.BlockSpec((1, EBLK), lambda i: (0, i)),
            pl.BlockSpec((1, EBLK), lambda i: (0, i)),
            pl.BlockSpec((1, EBLK), lambda i: (0, i)),
            pl.BlockSpec((1, EBLK), lambda i: (0, i)),
            pl.BlockSpec((2, 2), lambda i: (0, 0)),
        ],
        out_shape=[
            jax.ShapeDtypeStruct((1, n), jnp.float32),
            jax.ShapeDtypeStruct((1, n), jnp.float32),
            jax.ShapeDtypeStruct((1, n), jnp.float32),
            jax.ShapeDtypeStruct((1, n), jnp.float32),
            jax.ShapeDtypeStruct((2, 2), jnp.float32),
        ],
    )(step_i, edge_fts, W_edge, b_edge.reshape(2, 1), h32)


def _scatter_max16(acc, idx, val):
    def cond(act):
        return jnp.any(act)

    def body(act):
        cur = plsc.load_gather(acc, [idx])
        need = jnp.logical_and(act, val > cur)
        plsc.store_scatter(acc, [idx], val, mask=need)
        cur2 = plsc.load_gather(acc, [idx])
        return jnp.logical_and(need, val > cur2)

    act0 = val > plsc.load_gather(acc, [idx])
    lax.while_loop(cond, body, act0)


def _vfill(ref, n, value, dtype):
    def body(i, _):
        ref[pl.ds(i * L, L)] = jnp.full((L,), value, dtype)
        return 0

    lax.fori_loop(0, n // L, body, 0)


def _k1_partials(e_idx, l0, g0, l1, g1, batch_vec):
    @functools.partial(
        pl.kernel,
        out_type=(
            jax.ShapeDtypeStruct((NW, S0P), jnp.float32),
            jax.ShapeDtypeStruct((NW, S0P), jnp.float32),
            jax.ShapeDtypeStruct((NW, G), jnp.float32),
            jax.ShapeDtypeStruct((NW, G), jnp.float32),
        ),
        mesh=_sc_mesh(),
        compiler_params=_SC_PARAMS,
        scratch_types=[
            pltpu.VMEM((CH,), jnp.int32),
            pltpu.VMEM((CH,), jnp.float32),
            pltpu.VMEM((CH,), jnp.float32),
            pltpu.VMEM((CH,), jnp.float32),
            pltpu.VMEM((CH,), jnp.float32),
            pltpu.VMEM((N_NODES,), jnp.int32),
            pltpu.VMEM((S0P,), jnp.float32),
            pltpu.VMEM((S0P,), jnp.float32),
            pltpu.VMEM((G,), jnp.float32),
            pltpu.VMEM((G,), jnp.float32),
        ],
    )
    def k(idx_h, l0_h, g0_h, l1_h, g1_h, bv_h, m0p_h, g0p_h, m2p_h, g2p_h,
          idx_v, l0_v, g0_v, l1_v, g1_v, bv_v, m0a, g0a, m2a, g2a):
        wid = lax.axis_index("s") * 2 + lax.axis_index("c")
        base = wid * CH
        pltpu.sync_copy(idx_h.at[pl.ds(base, CH)], idx_v)
        pltpu.sync_copy(l0_h.at[pl.ds(base, CH)], l0_v)
        pltpu.sync_copy(g0_h.at[pl.ds(base, CH)], g0_v)
        pltpu.sync_copy(l1_h.at[pl.ds(base, CH)], l1_v)
        pltpu.sync_copy(g1_h.at[pl.ds(base, CH)], g1_v)
        pltpu.sync_copy(bv_h, bv_v)
        _vfill(m0a, S0P, NEG, jnp.float32)
        _vfill(g0a, S0P, 0.0, jnp.float32)
        _vfill(m2a, G, NEG, jnp.float32)
        _vfill(g2a, G, 0.0, jnp.float32)

        def step(j, _):
            sl = pl.ds(j * L, L)
            idx = idx_v[sl]
            _scatter_max16(m0a, idx, l0_v[sl])
            plsc.addupdate_scatter(g0a, [idx], g0_v[sl])
            idx2 = plsc.load_gather(bv_v, [idx])
            _scatter_max16(m2a, idx2, l1_v[sl])
            plsc.addupdate_scatter(g2a, [idx2], g1_v[sl])
            return 0

        lax.fori_loop(0, CH // L, step, 0)
        pltpu.sync_copy(m0a, m0p_h.at[wid])
        pltpu.sync_copy(g0a, g0p_h.at[wid])
        pltpu.sync_copy(m2a, m2p_h.at[wid])
        pltpu.sync_copy(g2a, g2p_h.at[wid])

    return k(e_idx, l0, g0, l1, g1, batch_vec)


def _c1_body(step_ref, tf_ref, m0p_ref, g0p_ref, m2p_ref, g2p_ref, nf_ref,
             w_ref, b_ref, h48_ref, bv_ref, m0f_ref, g0f_ref, m2f_ref,
             g2f_ref, states_ref, np_ref):
    m0f_ref[...] = jnp.max(m0p_ref[...], axis=0, keepdims=True)
    g0f_ref[...] = jnp.sum(g0p_ref[...], axis=0, keepdims=True)
    m2f_ref[...] = jnp.max(m2p_ref[...], axis=0, keepdims=True)
    g2f_ref[...] = jnp.sum(g2p_ref[...], axis=0, keepdims=True)

    step = step_ref[0]
    bi = lax.broadcasted_iota(jnp.int32, (48, 3), 0)
    bj = lax.broadcasted_iota(jnp.int32, (48, 3), 1)
    ohs = (bi == 3 * step + bj).astype(jnp.float32)
    gt = lax.dot_general(h48_ref[...], ohs,
                         (((1,), (0,)), ((), ())))

    logits = lax.dot_general(nf_ref[...], w_ref[...],
                             (((1,), (1,)), ((), ())))
    logits = logits + b_ref[...]
    bv = bv_ref[...]
    onehot = bv == lax.broadcasted_iota(jnp.int32, (N_NODES, G), 1)

    l0 = logits[:, 0:1]
    g0 = gt[:, 0:1]
    m_n = jnp.max(jnp.where(onehot, l0, NEG), axis=0, keepdims=True)
    gseg = jnp.sum(jnp.where(onehot, g0, 0.0), axis=0, keepdims=True)
    m_gath = jnp.sum(jnp.where(onehot, m_n, 0.0), axis=1, keepdims=True)
    denom = jnp.sum(jnp.where(onehot, jnp.exp(l0 - m_gath), 0.0),
                    axis=0, keepdims=True)
    loss_n0 = (-jnp.sum(g0 * l0) + jnp.sum(gseg * m_n) +
               jnp.sum(gseg * jnp.log(denom + 1e-20))) / G
    pred0 = (l0 >= m_gath).astype(jnp.float32)

    l1 = logits[:, 1:2]
    g1 = gt[:, 1:2]
    bce1 = jnp.sum(jnp.maximum(l1, 0.0) - l1 * g1 +
                   jnp.log1p(jnp.exp(-jnp.abs(l1)))) / N_NODES
    l2 = logits[:, 2:3]
    g2 = gt[:, 2:3]
    bce2 = jnp.sum(jnp.maximum(l2, 0.0) - l2 * g2 +
                   jnp.log1p(jnp.exp(-jnp.abs(l2)))) / N_NODES
    preds = jnp.concatenate(
        [pred0, (l1 > 0.0).astype(jnp.float32),
         (l2 > 0.0).astype(jnp.float32)], axis=1)
    states_ref[...] = jnp.where(tf_ref[0] != 0, gt, preds)

    n0count = jnp.sum(jnp.where(onehot[:, 0:1], 1.0, 0.0))
    loss_node = loss_n0 + bce1 + bce2
    ii = lax.broadcasted_iota(jnp.int32, (1, G), 1)
    np_ref[...] = jnp.where(ii == 0, loss_node,
                            jnp.where(ii == 1, n0count, 0.0))


def _c1(m0p, g0p, m2p, g2p, node_fts, W_node, b_node, h48, batch_vec,
        step_i, tf_i):
    return pl.pallas_call(
        _c1_body,
        grid=(1,),
        in_specs=[
            pl.BlockSpec(memory_space=pltpu.SMEM),
            pl.BlockSpec(memory_space=pltpu.SMEM),
            pl.BlockSpec((NW, S0P), lambda i: (0, 0)),
            pl.BlockSpec((NW, S0P), lambda i: (0, 0)),
            pl.BlockSpec((NW, G), lambda i: (0, 0)),
            pl.BlockSpec((NW, G), lambda i: (0, 0)),
            pl.BlockSpec((N_NODES, H), lambda i: (0, 0)),
            pl.BlockSpec((3, H), lambda i: (0, 0)),
            pl.BlockSpec((1, 3), lambda i: (0, 0)),
            pl.BlockSpec((N_NODES, 48), lambda i: (0, 0)),
            pl.BlockSpec((N_NODES, 1), lambda i: (0, 0)),
        ],
        out_specs=[
            pl.BlockSpec((1, S0P), lambda i: (0, 0)),
            pl.BlockSpec((1, S0P), lambda i: (0, 0)),
            pl.BlockSpec((1, G), lambda i: (0, 0)),
            pl.BlockSpec((1, G), lambda i: (0, 0)),
            pl.BlockSpec((N_NODES, 3), lambda i: (0, 0)),
            pl.BlockSpec((1, G), lambda i: (0, 0)),
        ],
        out_shape=[
            jax.ShapeDtypeStruct((1, S0P), jnp.float32),
            jax.ShapeDtypeStruct((1, S0P), jnp.float32),
            jax.ShapeDtypeStruct((1, G), jnp.float32),
            jax.ShapeDtypeStruct((1, G), jnp.float32),
            jax.ShapeDtypeStruct((N_NODES, 3), jnp.float32),
            jax.ShapeDtypeStruct((1, G), jnp.float32),
        ],
    )(step_i, tf_i, m0p, g0p, m2p, g2p, node_fts, W_node,
      b_node.reshape(1, 3), h48, batch_vec.reshape(N_NODES, 1))


def _k3_denoms_states(e_idx, l0, l1, g0, g1, batch_vec, m0f, m2f, tf16):
    @functools.partial(
        pl.kernel,
        out_type=(
            jax.ShapeDtypeStruct((2 * N_EDGES,), jnp.float32),
            jax.ShapeDtypeStruct((NW, S0P), jnp.float32),
            jax.ShapeDtypeStruct((NW, G), jnp.float32),
        ),
        mesh=_sc_mesh(),
        compiler_params=_SC_PARAMS,
        scratch_types=[
            pltpu.VMEM((CH,), jnp.int32),
            pltpu.VMEM((CH,), jnp.float32),
            pltpu.VMEM((CH,), jnp.float32),
            pltpu.VMEM((CH,), jnp.float32),
            pltpu.VMEM((CH,), jnp.float32),
            pltpu.VMEM((N_NODES,), jnp.int32),
            pltpu.VMEM((S0P,), jnp.float32),
            pltpu.VMEM((G,), jnp.float32),
            pltpu.VMEM((2 * CH,), jnp.float32),
            pltpu.VMEM((S0P,), jnp.float32),
            pltpu.VMEM((G,), jnp.float32),
            pltpu.VMEM((L,), jnp.int32),
        ],
    )
    def k(idx_h, l0_h, l1_h, g0_h, g1_h, bv_h, m0f_h, m2f_h, tf_h,
          st_h, d0p_h, d2p_h,
          idx_v, l0_v, l1_v, g0_v, g1_v, bv_v, m0f_v, m2f_v, st_v,
          d0a, d2a, tf_v):
        wid = lax.axis_index("s") * 2 + lax.axis_index("c")
        base = wid * CH
        pltpu.sync_copy(idx_h.at[pl.ds(base, CH)], idx_v)
        pltpu.sync_copy(l0_h.at[pl.ds(base, CH)], l0_v)
        pltpu.sync_copy(l1_h.at[pl.ds(base, CH)], l1_v)
        pltpu.sync_copy(g0_h.at[pl.ds(base, CH)], g0_v)
        pltpu.sync_copy(g1_h.at[pl.ds(base, CH)], g1_v)
        pltpu.sync_copy(bv_h, bv_v)
        pltpu.sync_copy(m0f_h, m0f_v)
        pltpu.sync_copy(m2f_h, m2f_v)
        pltpu.sync_copy(tf_h, tf_v)
        _vfill(d0a, S0P, 0.0, jnp.float32)
        _vfill(d2a, G, 0.0, jnp.float32)
        tfv = tf_v[pl.ds(0, L)] != 0
        lanes2 = 2 * lax.iota(jnp.int32, L)

        def step(j, _):
            sl = pl.ds(j * L, L)
            idx = idx_v[sl]
            v0 = l0_v[sl]
            m0g = plsc.load_gather(m0f_v, [idx])
            s0 = jnp.where(tfv, g0_v[sl], (v0 >= m0g).astype(jnp.float32))
            plsc.store_scatter(st_v, [j * (2 * L) + lanes2], s0)
            plsc.addupdate_scatter(d0a, [idx], jnp.exp(v0 - m0g))
            idx2 = plsc.load_gather(bv_v, [idx])
            v1 = l1_v[sl]
            m2g = plsc.load_gather(m2f_v, [idx2])
            s1 = jnp.where(tfv, g1_v[sl], (v1 >= m2g).astype(jnp.float32))
            plsc.store_scatter(st_v, [j * (2 * L) + lanes2 + 1], s1)
            plsc.addupdate_scatter(d2a, [idx2], jnp.exp(v1 - m2g))
            return 0

        lax.fori_loop(0, CH // L, step, 0)
        pltpu.sync_copy(st_v, st_h.at[pl.ds(2 * base, 2 * CH)])
        pltpu.sync_copy(d0a, d0p_h.at[wid])
        pltpu.sync_copy(d2a, d2p_h.at[wid])

    return k(e_idx, l0, l1, g0, g1, batch_vec, m0f, m2f, tf16)


def _c2_body(d0p_ref, d2p_ref, m0f_ref, g0f_ref, m2f_ref, g2f_ref,
             dots_ref, np_ref, out_ref):
    d0f = jnp.sum(d0p_ref[...], axis=0, keepdims=True)
    d2f = jnp.sum(d2p_ref[...], axis=0, keepdims=True)
    dots = dots_ref[...]
    dot0 = dots[0, 0]
    dot1 = dots[1, 1]
    npv = np_ref[...]
    ii = lax.broadcasted_iota(jnp.int32, (1, G), 1)
    loss_node = jnp.sum(jnp.where(ii == 0, npv, 0.0))
    n0count = jnp.sum(jnp.where(ii == 1, npv, 0.0))
    g0f = g0f_ref[...]
    m0f = m0f_ref[...]
    g2f = g2f_ref[...]
    m2f = m2f_ref[...]
    loss_a = (-dot0 + jnp.sum(g0f * m0f) +
              jnp.sum(g0f * jnp.log(d0f + 1e-20))) / N_NODES
    loss_b = n0count * (-dot1 + jnp.sum(g2f * m2f) +
                        jnp.sum(g2f * jnp.log(d2f + 1e-20))) / G
    out_ref[...] = jnp.full((1, 1), loss_node + loss_a + loss_b, jnp.float32)


def _c2(d0p, d2p, m0f, g0f, m2f, g2f, dots, npart):
    return pl.pallas_call(
        _c2_body,
        grid=(1,),
        in_specs=[
            pl.BlockSpec((NW, S0P), lambda i: (0, 0)),
            pl.BlockSpec((NW, G), lambda i: (0, 0)),
            pl.BlockSpec((1, S0P), lambda i: (0, 0)),
            pl.BlockSpec((1, S0P), lambda i: (0, 0)),
            pl.BlockSpec((1, G), lambda i: (0, 0)),
            pl.BlockSpec((1, G), lambda i: (0, 0)),
            pl.BlockSpec((2, 2), lambda i: (0, 0)),
            pl.BlockSpec((1, G), lambda i: (0, 0)),
        ],
        out_specs=pl.BlockSpec((1, 1), lambda i: (0, 0)),
        out_shape=jax.ShapeDtypeStruct((1, 1), jnp.float32),
    )(d0p, d2p, m0f, g0f, m2f, g2f, dots, npart)


def kernel(node_fts, edge_fts, node_hints, edge_hints, W_node, b_node, W_edge,
           b_edge, batch_vec, edge_index, processor_step, training_step,
           teacher_force):
    h32 = edge_hints.reshape(N_EDGES, 32)
    h48 = node_hints.reshape(N_NODES, 48)
    step_i = jnp.asarray(processor_step, jnp.int32).reshape(1)
    tf_i = jnp.asarray(teacher_force, jnp.int32).reshape(1)
    tf16 = jnp.broadcast_to(tf_i, (L,))
    e_idx = edge_index[0].astype(jnp.int32)
    batch_vec = batch_vec.astype(jnp.int32)

    l0, l1, g0, g1, dots = _edge_logits(edge_fts, W_edge, b_edge, h32, step_i)
    l0 = l0.reshape(N_EDGES)
    l1 = l1.reshape(N_EDGES)
    g0 = g0.reshape(N_EDGES)
    g1 = g1.reshape(N_EDGES)

    m0p, g0p, m2p, g2p = _k1_partials(e_idx, l0, g0, l1, g1, batch_vec)
    m0f, g0f, m2f, g2f, states_n, npart = _c1(
        m0p, g0p, m2p, g2p, node_fts, W_node, b_node, h48, batch_vec,
        step_i, tf_i)
    st_flat, d0p, d2p = _k3_denoms_states(
        e_idx, l0, l1, g0, g1, batch_vec, m0f.reshape(S0P), m2f.reshape(G),
        tf16)
    loss11 = _c2(d0p, d2p, m0f, g0f, m2f, g2f, dots, npart)

    loss = loss11[0, 0]
    states_e = st_flat.reshape(N_EDGES, 2)
    return (states_n, states_e, loss)


# speedup vs baseline: 47.1477x; 2.7720x over previous
import functools

import jax
import jax.numpy as jnp
from jax import lax
from jax.experimental import pallas as pl
from jax.experimental.pallas import tpu as pltpu
from jax.experimental.pallas import tpu_sc as plsc

N_NODES = 10000
N_EDGES = 320000
H = 128
G = 128
EBLK = 8192
S0P = 10112
NW = 32
CH = N_EDGES // NW
L = 16
NEG = -3.4e38

_SC_PARAMS = pltpu.CompilerParams(needs_layout_passes=False)


def _sc_mesh():
    return plsc.VectorSubcoreMesh(
        core_axis_name="c", subcore_axis_name="s", num_cores=2, num_subcores=16)


def _a_body(fts_ref, w_ref, b_ref, l0_ref, l1_ref):
    lg = lax.dot_general(w_ref[...], fts_ref[...],
                         (((1,), (1,)), ((), ())))
    lg = lg + b_ref[...]
    l0_ref[...] = lg[0]
    l1_ref[...] = lg[1]


def _edge_logits(edge_fts, W_edge, b_edge):
    return pl.pallas_call(
        _a_body,
        grid=((N_EDGES + EBLK - 1) // EBLK,),
        in_specs=[
            pl.BlockSpec((EBLK, H), lambda i: (i, 0)),
            pl.BlockSpec((2, H), lambda i: (0, 0)),
            pl.BlockSpec((2, 1), lambda i: (0, 0)),
        ],
        out_specs=[
            pl.BlockSpec((EBLK,), lambda i: (i,)),
            pl.BlockSpec((EBLK,), lambda i: (i,)),
        ],
        out_shape=[
            jax.ShapeDtypeStruct((N_EDGES,), jnp.float32),
            jax.ShapeDtypeStruct((N_EDGES,), jnp.float32),
        ],
    )(edge_fts, W_edge, b_edge.reshape(2, 1))


def _scatter_max16(acc, idx, val):
    def cond(act):
        return jnp.any(act)

    def body(act):
        cur = plsc.load_gather(acc, [idx])
        need = jnp.logical_and(act, val > cur)
        plsc.store_scatter(acc, [idx], val, mask=need)
        cur2 = plsc.load_gather(acc, [idx])
        return jnp.logical_and(need, val > cur2)

    act0 = val > plsc.load_gather(acc, [idx])
    lax.while_loop(cond, body, act0)


def _vfill(ref, n, value, dtype):
    def body(i, _):
        ref[pl.ds(i * L, L)] = jnp.full((L,), value, dtype)
        return 0

    lax.fori_loop(0, n // L, body, 0)


def _k1_partials(e_idx, l0, g0, l1, g1, batch_vec):
    @functools.partial(
        pl.kernel,
        out_type=(
            jax.ShapeDtypeStruct((NW, S0P), jnp.float32),
            jax.ShapeDtypeStruct((NW, S0P), jnp.float32),
            jax.ShapeDtypeStruct((NW, G), jnp.float32),
            jax.ShapeDtypeStruct((NW, G), jnp.float32),
            jax.ShapeDtypeStruct((NW, L), jnp.float32),
            jax.ShapeDtypeStruct((NW, L), jnp.float32),
        ),
        mesh=_sc_mesh(),
        compiler_params=_SC_PARAMS,
        scratch_types=[
            pltpu.VMEM((CH,), jnp.int32),
            pltpu.VMEM((CH,), jnp.float32),
            pltpu.VMEM((CH,), jnp.float32),
            pltpu.VMEM((CH,), jnp.float32),
            pltpu.VMEM((CH,), jnp.float32),
            pltpu.VMEM((N_NODES,), jnp.int32),
            pltpu.VMEM((S0P,), jnp.float32),
            pltpu.VMEM((S0P,), jnp.float32),
            pltpu.VMEM((G,), jnp.float32),
            pltpu.VMEM((G,), jnp.float32),
            pltpu.VMEM((L,), jnp.float32),
        ],
    )
    def k(idx_h, l0_h, g0_h, l1_h, g1_h, bv_h,
          m0p_h, g0p_h, m2p_h, g2p_h, dp0_h, dp1_h,
          idx_v, l0_v, g0_v, l1_v, g1_v, bv_v, m0a, g0a, m2a, g2a, dt_v):
        wid = lax.axis_index("s") * 2 + lax.axis_index("c")
        base = wid * CH
        pltpu.sync_copy(idx_h.at[pl.ds(base, CH)], idx_v)
        pltpu.sync_copy(l0_h.at[pl.ds(base, CH)], l0_v)
        pltpu.sync_copy(g0_h.at[pl.ds(base, CH)], g0_v)
        pltpu.sync_copy(l1_h.at[pl.ds(base, CH)], l1_v)
        pltpu.sync_copy(g1_h.at[pl.ds(base, CH)], g1_v)
        pltpu.sync_copy(bv_h, bv_v)
        _vfill(m0a, S0P, NEG, jnp.float32)
        _vfill(g0a, S0P, 0.0, jnp.float32)
        _vfill(m2a, G, NEG, jnp.float32)
        _vfill(g2a, G, 0.0, jnp.float32)

        def step(j, carry):
            dv0, dv1 = carry
            sl = pl.ds(j * L, L)
            idx = idx_v[sl]
            lv0 = l0_v[sl]
            gv0 = g0_v[sl]
            _scatter_max16(m0a, idx, lv0)
            plsc.addupdate_scatter(g0a, [idx], gv0)
            idx2 = plsc.load_gather(bv_v, [idx])
            lv1 = l1_v[sl]
            gv1 = g1_v[sl]
            _scatter_max16(m2a, idx2, lv1)
            plsc.addupdate_scatter(g2a, [idx2], gv1)
            return (dv0 + gv0 * lv0, dv1 + gv1 * lv1)

        zero = jnp.zeros((L,), jnp.float32)
        dv0, dv1 = lax.fori_loop(0, CH // L, step, (zero, zero))
        pltpu.sync_copy(m0a, m0p_h.at[wid])
        pltpu.sync_copy(g0a, g0p_h.at[wid])
        pltpu.sync_copy(m2a, m2p_h.at[wid])
        pltpu.sync_copy(g2a, g2p_h.at[wid])
        dt_v[pl.ds(0, L)] = dv0
        pltpu.sync_copy(dt_v, dp0_h.at[wid])
        dt_v[pl.ds(0, L)] = dv1
        pltpu.sync_copy(dt_v, dp1_h.at[wid])

    return k(e_idx, l0, g0, l1, g1, batch_vec)


_CB = 2500


def _c1a_chunk(c, nf_ref, w_ref, b_ref, g0_ref, g1_ref, g2_ref, bv_ref):
    sl = pl.ds(c * _CB, _CB)
    x = nf_ref[sl, :]
    logits = lax.dot_general(x, w_ref[...],
                             (((1,), (1,)), ((), ())))
    logits = logits + b_ref[...]
    gt = jnp.concatenate([g0_ref[sl, :], g1_ref[sl, :], g2_ref[sl, :]],
                         axis=1)
    bv = bv_ref[sl, :]
    onehot = bv == lax.broadcasted_iota(jnp.int32, (_CB, G), 1)
    return logits, gt, onehot


def _c1a_body(tf_ref, nf_ref, w_ref, b_ref, g0_ref, g1_ref, g2_ref, bv_ref,
              states_ref, np_ref):
    def ph1(c, carry):
        m_n, gseg, dotn, bce1, bce2, n0c = carry
        logits, gt, onehot = _c1a_chunk(c, nf_ref, w_ref, b_ref, g0_ref,
                                        g1_ref, g2_ref, bv_ref)
        l0 = logits[:, 0:1]
        g0 = gt[:, 0:1]
        m_n = jnp.maximum(m_n, jnp.max(jnp.where(onehot, l0, NEG), axis=0,
                                       keepdims=True))
        gseg = gseg + jnp.sum(jnp.where(onehot, g0, 0.0), axis=0,
                              keepdims=True)
        dotn = dotn + jnp.sum(g0 * l0)
        l1 = logits[:, 1:2]
        g1 = gt[:, 1:2]
        bce1 = bce1 + jnp.sum(jnp.maximum(l1, 0.0) - l1 * g1 +
                              jnp.log1p(jnp.exp(-jnp.abs(l1))))
        l2 = logits[:, 2:3]
        g2 = gt[:, 2:3]
        bce2 = bce2 + jnp.sum(jnp.maximum(l2, 0.0) - l2 * g2 +
                              jnp.log1p(jnp.exp(-jnp.abs(l2))))
        n0c = n0c + jnp.sum(jnp.where(onehot[:, 0:1], 1.0, 0.0))
        return m_n, gseg, dotn, bce1, bce2, n0c

    init = (jnp.full((1, G), NEG, jnp.float32),
            jnp.zeros((1, G), jnp.float32),
            jnp.float32(0.0), jnp.float32(0.0), jnp.float32(0.0),
            jnp.float32(0.0))
    m_n, gseg, dotn, bce1, bce2, n0c = lax.fori_loop(
        0, N_NODES // _CB, ph1, init)

    def ph2(c, denom):
        logits, gt, onehot = _c1a_chunk(c, nf_ref, w_ref, b_ref, g0_ref,
                                        g1_ref, g2_ref, bv_ref)
        l0 = logits[:, 0:1]
        m_gath = jnp.sum(jnp.where(onehot, m_n, 0.0), axis=1, keepdims=True)
        denom = denom + jnp.sum(jnp.where(onehot, jnp.exp(l0 - m_gath), 0.0),
                                axis=0, keepdims=True)
        preds = jnp.concatenate(
            [(l0 >= m_gath).astype(jnp.float32),
             (logits[:, 1:2] > 0.0).astype(jnp.float32),
             (logits[:, 2:3] > 0.0).astype(jnp.float32)], axis=1)
        states_ref[pl.ds(c * _CB, _CB), :] = jnp.where(tf_ref[0] != 0, gt,
                                                       preds)
        return denom

    denom = lax.fori_loop(0, N_NODES // _CB, ph2,
                          jnp.zeros((1, G), jnp.float32))

    loss_n0 = (-dotn + jnp.sum(gseg * m_n) +
               jnp.sum(gseg * jnp.log(denom + 1e-20))) / G
    loss_node = loss_n0 + bce1 / N_NODES + bce2 / N_NODES
    ii = lax.broadcasted_iota(jnp.int32, (1, G), 1)
    np_ref[...] = jnp.where(ii == 0, loss_node,
                            jnp.where(ii == 1, n0c, 0.0))


def _c1a(node_fts, W_node, b_node, gtn0, gtn1, gtn2, batch_vec, tf_i):
    return pl.pallas_call(
        _c1a_body,
        grid=(1,),
        in_specs=[
            pl.BlockSpec(memory_space=pltpu.SMEM),
            pl.BlockSpec((N_NODES, H), lambda i: (0, 0)),
            pl.BlockSpec((3, H), lambda i: (0, 0)),
            pl.BlockSpec((1, 3), lambda i: (0, 0)),
            pl.BlockSpec((N_NODES, 1), lambda i: (0, 0)),
            pl.BlockSpec((N_NODES, 1), lambda i: (0, 0)),
            pl.BlockSpec((N_NODES, 1), lambda i: (0, 0)),
            pl.BlockSpec((N_NODES, 1), lambda i: (0, 0)),
        ],
        out_specs=[
            pl.BlockSpec((N_NODES, 3), lambda i: (0, 0)),
            pl.BlockSpec((1, G), lambda i: (0, 0)),
        ],
        out_shape=[
            jax.ShapeDtypeStruct((N_NODES, 3), jnp.float32),
            jax.ShapeDtypeStruct((1, G), jnp.float32),
        ],
    )(tf_i, node_fts, W_node, b_node.reshape(1, 3), gtn0, gtn1, gtn2,
      batch_vec.reshape(N_NODES, 1))


def _c1b_body(m0p_ref, g0p_ref, m2p_ref, g2p_ref,
              m0f_ref, g0f_ref, m2f_ref, g2f_ref):
    m0f_ref[...] = jnp.max(m0p_ref[...], axis=0, keepdims=True)
    g0f_ref[...] = jnp.sum(g0p_ref[...], axis=0, keepdims=True)
    m2f_ref[...] = jnp.max(m2p_ref[...], axis=0, keepdims=True)
    g2f_ref[...] = jnp.sum(g2p_ref[...], axis=0, keepdims=True)


def _c1b(m0p, g0p, m2p, g2p):
    return pl.pallas_call(
        _c1b_body,
        grid=(1,),
        in_specs=[
            pl.BlockSpec((NW, S0P), lambda i: (0, 0)),
            pl.BlockSpec((NW, S0P), lambda i: (0, 0)),
            pl.BlockSpec((NW, G), lambda i: (0, 0)),
            pl.BlockSpec((NW, G), lambda i: (0, 0)),
        ],
        out_specs=[
            pl.BlockSpec((1, S0P), lambda i: (0, 0)),
            pl.BlockSpec((1, S0P), lambda i: (0, 0)),
            pl.BlockSpec((1, G), lambda i: (0, 0)),
            pl.BlockSpec((1, G), lambda i: (0, 0)),
        ],
        out_shape=[
            jax.ShapeDtypeStruct((1, S0P), jnp.float32),
            jax.ShapeDtypeStruct((1, S0P), jnp.float32),
            jax.ShapeDtypeStruct((1, G), jnp.float32),
            jax.ShapeDtypeStruct((1, G), jnp.float32),
        ],
    )(m0p, g0p, m2p, g2p)


def _k3_denoms_states(e_idx, l0, l1, g0, g1, batch_vec, m0f, m2f, tf16):
    @functools.partial(
        pl.kernel,
        out_type=(
            jax.ShapeDtypeStruct((N_EDGES,), jnp.float32),
            jax.ShapeDtypeStruct((N_EDGES,), jnp.float32),
            jax.ShapeDtypeStruct((NW, S0P), jnp.float32),
            jax.ShapeDtypeStruct((NW, G), jnp.float32),
        ),
        mesh=_sc_mesh(),
        compiler_params=_SC_PARAMS,
        scratch_types=[
            pltpu.VMEM((CH,), jnp.int32),
            pltpu.VMEM((CH,), jnp.float32),
            pltpu.VMEM((CH,), jnp.float32),
            pltpu.VMEM((CH,), jnp.float32),
            pltpu.VMEM((CH,), jnp.float32),
            pltpu.VMEM((N_NODES,), jnp.int32),
            pltpu.VMEM((S0P,), jnp.float32),
            pltpu.VMEM((G,), jnp.float32),
            pltpu.VMEM((CH,), jnp.float32),
            pltpu.VMEM((CH,), jnp.float32),
            pltpu.VMEM((S0P,), jnp.float32),
            pltpu.VMEM((G,), jnp.float32),
            pltpu.VMEM((L,), jnp.int32),
        ],
    )
    def k(idx_h, l0_h, l1_h, g0_h, g1_h, bv_h, m0f_h, m2f_h, tf_h,
          s0_h, s1_h, d0p_h, d2p_h,
          idx_v, l0_v, l1_v, g0_v, g1_v, bv_v, m0f_v, m2f_v, s0_v, s1_v,
          d0a, d2a, tf_v):
        wid = lax.axis_index("s") * 2 + lax.axis_index("c")
        base = wid * CH
        pltpu.sync_copy(idx_h.at[pl.ds(base, CH)], idx_v)
        pltpu.sync_copy(l0_h.at[pl.ds(base, CH)], l0_v)
        pltpu.sync_copy(l1_h.at[pl.ds(base, CH)], l1_v)
        pltpu.sync_copy(g0_h.at[pl.ds(base, CH)], g0_v)
        pltpu.sync_copy(g1_h.at[pl.ds(base, CH)], g1_v)
        pltpu.sync_copy(bv_h, bv_v)
        pltpu.sync_copy(m0f_h, m0f_v)
        pltpu.sync_copy(m2f_h, m2f_v)
        pltpu.sync_copy(tf_h, tf_v)
        _vfill(d0a, S0P, 0.0, jnp.float32)
        _vfill(d2a, G, 0.0, jnp.float32)
        tfv = tf_v[pl.ds(0, L)] != 0

        def step(j, _):
            sl = pl.ds(j * L, L)
            idx = idx_v[sl]
            v0 = l0_v[sl]
            m0g = plsc.load_gather(m0f_v, [idx])
            s0_v[sl] = jnp.where(tfv, g0_v[sl],
                                 (v0 >= m0g).astype(jnp.float32))
            plsc.addupdate_scatter(d0a, [idx], jnp.exp(v0 - m0g))
            idx2 = plsc.load_gather(bv_v, [idx])
            v1 = l1_v[sl]
            m2g = plsc.load_gather(m2f_v, [idx2])
            s1_v[sl] = jnp.where(tfv, g1_v[sl],
                                 (v1 >= m2g).astype(jnp.float32))
            plsc.addupdate_scatter(d2a, [idx2], jnp.exp(v1 - m2g))
            return 0

        lax.fori_loop(0, CH // L, step, 0)
        pltpu.sync_copy(s0_v, s0_h.at[pl.ds(base, CH)])
        pltpu.sync_copy(s1_v, s1_h.at[pl.ds(base, CH)])
        pltpu.sync_copy(d0a, d0p_h.at[wid])
        pltpu.sync_copy(d2a, d2p_h.at[wid])

    return k(e_idx, l0, l1, g0, g1, batch_vec, m0f, m2f, tf16)


def _c2_body(d0p_ref, d2p_ref, m0f_ref, g0f_ref, m2f_ref, g2f_ref,
             dp0_ref, dp1_ref, np_ref, out_ref):
    d0f = jnp.sum(d0p_ref[...], axis=0, keepdims=True)
    d2f = jnp.sum(d2p_ref[...], axis=0, keepdims=True)
    dot0 = jnp.sum(dp0_ref[...])
    dot1 = jnp.sum(dp1_ref[...])
    npv = np_ref[...]
    ii = lax.broadcasted_iota(jnp.int32, (1, G), 1)
    loss_node = jnp.sum(jnp.where(ii == 0, npv, 0.0))
    n0count = jnp.sum(jnp.where(ii == 1, npv, 0.0))
    g0f = g0f_ref[...]
    m0f = m0f_ref[...]
    g2f = g2f_ref[...]
    m2f = m2f_ref[...]
    loss_a = (-dot0 + jnp.sum(g0f * m0f) +
              jnp.sum(g0f * jnp.log(d0f + 1e-20))) / N_NODES
    loss_b = n0count * (-dot1 + jnp.sum(g2f * m2f) +
                        jnp.sum(g2f * jnp.log(d2f + 1e-20))) / G
    out_ref[...] = jnp.full((1, 1), loss_node + loss_a + loss_b, jnp.float32)


def _c2(d0p, d2p, m0f, g0f, m2f, g2f, dp0, dp1, npart):
    return pl.pallas_call(
        _c2_body,
        grid=(1,),
        in_specs=[
            pl.BlockSpec((NW, S0P), lambda i: (0, 0)),
            pl.BlockSpec((NW, G), lambda i: (0, 0)),
            pl.BlockSpec((1, S0P), lambda i: (0, 0)),
            pl.BlockSpec((1, S0P), lambda i: (0, 0)),
            pl.BlockSpec((1, G), lambda i: (0, 0)),
            pl.BlockSpec((1, G), lambda i: (0, 0)),
            pl.BlockSpec((NW, L), lambda i: (0, 0)),
            pl.BlockSpec((NW, L), lambda i: (0, 0)),
            pl.BlockSpec((1, G), lambda i: (0, 0)),
        ],
        out_specs=pl.BlockSpec((1, 1), lambda i: (0, 0)),
        out_shape=jax.ShapeDtypeStruct((1, 1), jnp.float32),
    )(d0p, d2p, m0f, g0f, m2f, g2f, dp0, dp1, npart)


def kernel(node_fts, edge_fts, node_hints, edge_hints, W_node, b_node, W_edge,
           b_edge, batch_vec, edge_index, processor_step, training_step,
           teacher_force):
    step = jnp.asarray(processor_step, jnp.int32)
    tf_i = jnp.asarray(teacher_force, jnp.int32).reshape(1)
    tf16 = jnp.broadcast_to(tf_i, (L,))
    e_idx = edge_index[0].astype(jnp.int32)
    batch_vec = batch_vec.astype(jnp.int32)

    g0 = lax.dynamic_slice(edge_hints, (0, step, 0),
                           (N_EDGES, 1, 1)).reshape(N_EDGES)
    g1 = lax.dynamic_slice(edge_hints, (0, step, 1),
                           (N_EDGES, 1, 1)).reshape(N_EDGES)
    gtn = [lax.dynamic_slice(node_hints, (0, step, k),
                             (N_NODES, 1, 1)).reshape(N_NODES, 1)
           for k in range(3)]

    l0, l1 = _edge_logits(edge_fts, W_edge, b_edge)

    m0p, g0p, m2p, g2p, dp0, dp1 = _k1_partials(
        e_idx, l0, g0, l1, g1, batch_vec)
    states_n, npart = _c1a(node_fts, W_node, b_node, gtn[0], gtn[1], gtn[2],
                           batch_vec, tf_i)
    m0f, g0f, m2f, g2f = _c1b(m0p, g0p, m2p, g2p)
    s0, s1, d0p, d2p = _k3_denoms_states(
        e_idx, l0, l1, g0, g1, batch_vec, m0f.reshape(S0P), m2f.reshape(G),
        tf16)
    loss11 = _c2(d0p, d2p, m0f, g0f, m2f, g2f, dp0, dp1, npart)

    loss = loss11[0, 0]
    states_e = jnp.stack([s0, s1], axis=-1)
    return (states_n, states_e, loss)


# speedup vs baseline: 53.1293x; 1.1269x over previous
import functools

import jax
import jax.numpy as jnp
from jax import lax
from jax.experimental import pallas as pl
from jax.experimental.pallas import tpu as pltpu
from jax.experimental.pallas import tpu_sc as plsc

N_NODES = 10000
N_EDGES = 320000
H = 128
G = 128
EBLK = 8192
S0P = 10112
NW = 32
CH = N_EDGES // NW
L = 16
NEG = -3.4e38

_SC_PARAMS = pltpu.CompilerParams(needs_layout_passes=False)


def _sc_mesh():
    return plsc.VectorSubcoreMesh(
        core_axis_name="c", subcore_axis_name="s", num_cores=2, num_subcores=16)


def _a_body(fts_ref, w_ref, b_ref, ei_ref, l0_ref, l1_ref, idx_ref):
    lg = lax.dot_general(w_ref[...], fts_ref[...],
                         (((1,), (1,)), ((), ())))
    lg = lg + b_ref[...]
    l0_ref[...] = lg[0]
    l1_ref[...] = lg[1]
    idx_ref[...] = ei_ref[0]


def _edge_logits(edge_fts, W_edge, b_edge, edge_index):
    return pl.pallas_call(
        _a_body,
        grid=((N_EDGES + EBLK - 1) // EBLK,),
        in_specs=[
            pl.BlockSpec((EBLK, H), lambda i: (i, 0)),
            pl.BlockSpec((2, H), lambda i: (0, 0)),
            pl.BlockSpec((2, 1), lambda i: (0, 0)),
            pl.BlockSpec((2, EBLK), lambda i: (0, i)),
        ],
        out_specs=[
            pl.BlockSpec((EBLK,), lambda i: (i,)),
            pl.BlockSpec((EBLK,), lambda i: (i,)),
            pl.BlockSpec((EBLK,), lambda i: (i,)),
        ],
        out_shape=[
            jax.ShapeDtypeStruct((N_EDGES,), jnp.float32),
            jax.ShapeDtypeStruct((N_EDGES,), jnp.float32),
            jax.ShapeDtypeStruct((N_EDGES,), jnp.int32),
        ],
    )(edge_fts, W_edge, b_edge.reshape(2, 1), edge_index)


def _scatter_max16(acc, idx, val):
    def cond(act):
        return jnp.any(act)

    def body(act):
        cur = plsc.load_gather(acc, [idx])
        need = jnp.logical_and(act, val > cur)
        plsc.store_scatter(acc, [idx], val, mask=need)
        cur2 = plsc.load_gather(acc, [idx])
        return jnp.logical_and(need, val > cur2)

    act0 = val > plsc.load_gather(acc, [idx])
    lax.while_loop(cond, body, act0)


def _scatter_max16_pair(acc_a, idx_a, val_a, acc_b, idx_b, val_b):
    def cond(st):
        aa, ab = st
        return jnp.any(jnp.logical_or(aa, ab))

    def body(st):
        aa, ab = st
        cura = plsc.load_gather(acc_a, [idx_a])
        needa = jnp.logical_and(aa, val_a > cura)
        plsc.store_scatter(acc_a, [idx_a], val_a, mask=needa)
        curb = plsc.load_gather(acc_b, [idx_b])
        needb = jnp.logical_and(ab, val_b > curb)
        plsc.store_scatter(acc_b, [idx_b], val_b, mask=needb)
        cura2 = plsc.load_gather(acc_a, [idx_a])
        curb2 = plsc.load_gather(acc_b, [idx_b])
        return (jnp.logical_and(needa, val_a > cura2),
                jnp.logical_and(needb, val_b > curb2))

    aa0 = val_a > plsc.load_gather(acc_a, [idx_a])
    ab0 = val_b > plsc.load_gather(acc_b, [idx_b])
    lax.while_loop(cond, body, (aa0, ab0))


def _vfill(ref, n, value, dtype):
    def body(i, _):
        ref[pl.ds(i * L, L)] = jnp.full((L,), value, dtype)
        return 0

    lax.fori_loop(0, n // L, body, 0)


def _k1_partials(e_idx, l0, g0, l1, g1, batch_vec):
    @functools.partial(
        pl.kernel,
        out_type=(
            jax.ShapeDtypeStruct((NW, S0P), jnp.float32),
            jax.ShapeDtypeStruct((NW, S0P), jnp.float32),
            jax.ShapeDtypeStruct((NW, G), jnp.float32),
            jax.ShapeDtypeStruct((NW, G), jnp.float32),
            jax.ShapeDtypeStruct((NW, L), jnp.float32),
            jax.ShapeDtypeStruct((NW, L), jnp.float32),
        ),
        mesh=_sc_mesh(),
        compiler_params=_SC_PARAMS,
        scratch_types=[
            pltpu.VMEM((CH,), jnp.int32),
            pltpu.VMEM((CH,), jnp.float32),
            pltpu.VMEM((CH,), jnp.float32),
            pltpu.VMEM((CH,), jnp.float32),
            pltpu.VMEM((CH,), jnp.float32),
            pltpu.VMEM((N_NODES,), jnp.int32),
            pltpu.VMEM((S0P,), jnp.float32),
            pltpu.VMEM((S0P,), jnp.float32),
            pltpu.VMEM((G,), jnp.float32),
            pltpu.VMEM((G,), jnp.float32),
            pltpu.VMEM((L,), jnp.float32),
        ],
    )
    def k(idx_h, l0_h, g0_h, l1_h, g1_h, bv_h,
          m0p_h, g0p_h, m2p_h, g2p_h, dp0_h, dp1_h,
          idx_v, l0_v, g0_v, l1_v, g1_v, bv_v, m0a, g0a, m2a, g2a, dt_v):
        wid = lax.axis_index("s") * 2 + lax.axis_index("c")
        base = wid * CH
        pltpu.sync_copy(idx_h.at[pl.ds(base, CH)], idx_v)
        pltpu.sync_copy(l0_h.at[pl.ds(base, CH)], l0_v)
        pltpu.sync_copy(g0_h.at[pl.ds(base, CH)], g0_v)
        pltpu.sync_copy(l1_h.at[pl.ds(base, CH)], l1_v)
        pltpu.sync_copy(g1_h.at[pl.ds(base, CH)], g1_v)
        pltpu.sync_copy(bv_h, bv_v)
        _vfill(m0a, S0P, NEG, jnp.float32)
        _vfill(g0a, S0P, 0.0, jnp.float32)
        _vfill(m2a, G, NEG, jnp.float32)
        _vfill(g2a, G, 0.0, jnp.float32)

        def step(j, carry):
            dv0, dv1 = carry
            sl = pl.ds(j * L, L)
            idx = idx_v[sl]
            lv0 = l0_v[sl]
            gv0 = g0_v[sl]
            idx2 = plsc.load_gather(bv_v, [idx])
            lv1 = l1_v[sl]
            gv1 = g1_v[sl]
            plsc.addupdate_scatter(g0a, [idx], gv0)
            plsc.addupdate_scatter(g2a, [idx2], gv1)
            _scatter_max16_pair(m0a, idx, lv0, m2a, idx2, lv1)
            return (dv0 + gv0 * lv0, dv1 + gv1 * lv1)

        zero = jnp.zeros((L,), jnp.float32)
        dv0, dv1 = lax.fori_loop(0, CH // L, step, (zero, zero))
        pltpu.sync_copy(m0a, m0p_h.at[wid])
        pltpu.sync_copy(g0a, g0p_h.at[wid])
        pltpu.sync_copy(m2a, m2p_h.at[wid])
        pltpu.sync_copy(g2a, g2p_h.at[wid])
        dt_v[pl.ds(0, L)] = dv0
        pltpu.sync_copy(dt_v, dp0_h.at[wid])
        dt_v[pl.ds(0, L)] = dv1
        pltpu.sync_copy(dt_v, dp1_h.at[wid])

    return k(e_idx, l0, g0, l1, g1, batch_vec)


_CB = 2500


def _c1a_chunk(c, nf_ref, w_ref, b_ref, g0_ref, g1_ref, g2_ref, bv_ref):
    sl = pl.ds(c * _CB, _CB)
    x = nf_ref[sl, :]
    logits = lax.dot_general(x, w_ref[...],
                             (((1,), (1,)), ((), ())))
    logits = logits + b_ref[...]
    gt = jnp.concatenate([g0_ref[sl, :], g1_ref[sl, :], g2_ref[sl, :]],
                         axis=1)
    bv = bv_ref[sl, :]
    onehot = bv == lax.broadcasted_iota(jnp.int32, (_CB, G), 1)
    return logits, gt, onehot


def _c1a_body(tf_ref, nf_ref, w_ref, b_ref, g0_ref, g1_ref, g2_ref, bv_ref,
              states_ref, np_ref):
    def ph1(c, carry):
        m_n, gseg, dotn, bce1, bce2, n0c = carry
        logits, gt, onehot = _c1a_chunk(c, nf_ref, w_ref, b_ref, g0_ref,
                                        g1_ref, g2_ref, bv_ref)
        l0 = logits[:, 0:1]
        g0 = gt[:, 0:1]
        m_n = jnp.maximum(m_n, jnp.max(jnp.where(onehot, l0, NEG), axis=0,
                                       keepdims=True))
        gseg = gseg + jnp.sum(jnp.where(onehot, g0, 0.0), axis=0,
                              keepdims=True)
        dotn = dotn + jnp.sum(g0 * l0)
        l1 = logits[:, 1:2]
        g1 = gt[:, 1:2]
        bce1 = bce1 + jnp.sum(jnp.maximum(l1, 0.0) - l1 * g1 +
                              jnp.log1p(jnp.exp(-jnp.abs(l1))))
        l2 = logits[:, 2:3]
        g2 = gt[:, 2:3]
        bce2 = bce2 + jnp.sum(jnp.maximum(l2, 0.0) - l2 * g2 +
                              jnp.log1p(jnp.exp(-jnp.abs(l2))))
        n0c = n0c + jnp.sum(jnp.where(onehot[:, 0:1], 1.0, 0.0))
        return m_n, gseg, dotn, bce1, bce2, n0c

    init = (jnp.full((1, G), NEG, jnp.float32),
            jnp.zeros((1, G), jnp.float32),
            jnp.float32(0.0), jnp.float32(0.0), jnp.float32(0.0),
            jnp.float32(0.0))
    m_n, gseg, dotn, bce1, bce2, n0c = lax.fori_loop(
        0, N_NODES // _CB, ph1, init)

    def ph2(c, denom):
        logits, gt, onehot = _c1a_chunk(c, nf_ref, w_ref, b_ref, g0_ref,
                                        g1_ref, g2_ref, bv_ref)
        l0 = logits[:, 0:1]
        m_gath = jnp.sum(jnp.where(onehot, m_n, 0.0), axis=1, keepdims=True)
        denom = denom + jnp.sum(jnp.where(onehot, jnp.exp(l0 - m_gath), 0.0),
                                axis=0, keepdims=True)
        preds = jnp.concatenate(
            [(l0 >= m_gath).astype(jnp.float32),
             (logits[:, 1:2] > 0.0).astype(jnp.float32),
             (logits[:, 2:3] > 0.0).astype(jnp.float32)], axis=1)
        states_ref[pl.ds(c * _CB, _CB), :] = jnp.where(tf_ref[0] != 0, gt,
                                                       preds)
        return denom

    denom = lax.fori_loop(0, N_NODES // _CB, ph2,
                          jnp.zeros((1, G), jnp.float32))

    loss_n0 = (-dotn + jnp.sum(gseg * m_n) +
               jnp.sum(gseg * jnp.log(denom + 1e-20))) / G
    loss_node = loss_n0 + bce1 / N_NODES + bce2 / N_NODES
    ii = lax.broadcasted_iota(jnp.int32, (1, G), 1)
    np_ref[...] = jnp.where(ii == 0, loss_node,
                            jnp.where(ii == 1, n0c, 0.0))


def _c1a(node_fts, W_node, b_node, gtn0, gtn1, gtn2, batch_vec, tf_i):
    return pl.pallas_call(
        _c1a_body,
        grid=(1,),
        in_specs=[
            pl.BlockSpec(memory_space=pltpu.SMEM),
            pl.BlockSpec((N_NODES, H), lambda i: (0, 0)),
            pl.BlockSpec((3, H), lambda i: (0, 0)),
            pl.BlockSpec((1, 3), lambda i: (0, 0)),
            pl.BlockSpec((N_NODES, 1), lambda i: (0, 0)),
            pl.BlockSpec((N_NODES, 1), lambda i: (0, 0)),
            pl.BlockSpec((N_NODES, 1), lambda i: (0, 0)),
            pl.BlockSpec((N_NODES, 1), lambda i: (0, 0)),
        ],
        out_specs=[
            pl.BlockSpec((N_NODES, 3), lambda i: (0, 0)),
            pl.BlockSpec((1, G), lambda i: (0, 0)),
        ],
        out_shape=[
            jax.ShapeDtypeStruct((N_NODES, 3), jnp.float32),
            jax.ShapeDtypeStruct((1, G), jnp.float32),
        ],
    )(tf_i, node_fts, W_node, b_node.reshape(1, 3), gtn0, gtn1, gtn2,
      batch_vec.reshape(N_NODES, 1))


def _c1b_body(m0p_ref, g0p_ref, m2p_ref, g2p_ref,
              m0f_ref, g0f_ref, m2f_ref, g2f_ref):
    m0f_ref[...] = jnp.max(m0p_ref[...], axis=0, keepdims=True)
    g0f_ref[...] = jnp.sum(g0p_ref[...], axis=0, keepdims=True)
    m2f_ref[...] = jnp.max(m2p_ref[...], axis=0, keepdims=True)
    g2f_ref[...] = jnp.sum(g2p_ref[...], axis=0, keepdims=True)


def _c1b(m0p, g0p, m2p, g2p):
    return pl.pallas_call(
        _c1b_body,
        grid=(1,),
        in_specs=[
            pl.BlockSpec((NW, S0P), lambda i: (0, 0)),
            pl.BlockSpec((NW, S0P), lambda i: (0, 0)),
            pl.BlockSpec((NW, G), lambda i: (0, 0)),
            pl.BlockSpec((NW, G), lambda i: (0, 0)),
        ],
        out_specs=[
            pl.BlockSpec((1, S0P), lambda i: (0, 0)),
            pl.BlockSpec((1, S0P), lambda i: (0, 0)),
            pl.BlockSpec((1, G), lambda i: (0, 0)),
            pl.BlockSpec((1, G), lambda i: (0, 0)),
        ],
        out_shape=[
            jax.ShapeDtypeStruct((1, S0P), jnp.float32),
            jax.ShapeDtypeStruct((1, S0P), jnp.float32),
            jax.ShapeDtypeStruct((1, G), jnp.float32),
            jax.ShapeDtypeStruct((1, G), jnp.float32),
        ],
    )(m0p, g0p, m2p, g2p)


def _k3_denoms_states(e_idx, l0, l1, g0, g1, batch_vec, m0f, m2f, tf16):
    @functools.partial(
        pl.kernel,
        out_type=(
            jax.ShapeDtypeStruct((N_EDGES,), jnp.float32),
            jax.ShapeDtypeStruct((N_EDGES,), jnp.float32),
            jax.ShapeDtypeStruct((NW, S0P), jnp.float32),
            jax.ShapeDtypeStruct((NW, G), jnp.float32),
        ),
        mesh=_sc_mesh(),
        compiler_params=_SC_PARAMS,
        scratch_types=[
            pltpu.VMEM((CH,), jnp.int32),
            pltpu.VMEM((CH,), jnp.float32),
            pltpu.VMEM((CH,), jnp.float32),
            pltpu.VMEM((CH,), jnp.float32),
            pltpu.VMEM((CH,), jnp.float32),
            pltpu.VMEM((N_NODES,), jnp.int32),
            pltpu.VMEM((S0P,), jnp.float32),
            pltpu.VMEM((G,), jnp.float32),
            pltpu.VMEM((CH,), jnp.float32),
            pltpu.VMEM((CH,), jnp.float32),
            pltpu.VMEM((S0P,), jnp.float32),
            pltpu.VMEM((G,), jnp.float32),
            pltpu.VMEM((L,), jnp.int32),
        ],
    )
    def k(idx_h, l0_h, l1_h, g0_h, g1_h, bv_h, m0f_h, m2f_h, tf_h,
          s0_h, s1_h, d0p_h, d2p_h,
          idx_v, l0_v, l1_v, g0_v, g1_v, bv_v, m0f_v, m2f_v, s0_v, s1_v,
          d0a, d2a, tf_v):
        wid = lax.axis_index("s") * 2 + lax.axis_index("c")
        base = wid * CH
        pltpu.sync_copy(idx_h.at[pl.ds(base, CH)], idx_v)
        pltpu.sync_copy(l0_h.at[pl.ds(base, CH)], l0_v)
        pltpu.sync_copy(l1_h.at[pl.ds(base, CH)], l1_v)
        pltpu.sync_copy(g0_h.at[pl.ds(base, CH)], g0_v)
        pltpu.sync_copy(g1_h.at[pl.ds(base, CH)], g1_v)
        pltpu.sync_copy(bv_h, bv_v)
        pltpu.sync_copy(m0f_h, m0f_v)
        pltpu.sync_copy(m2f_h, m2f_v)
        pltpu.sync_copy(tf_h, tf_v)
        _vfill(d0a, S0P, 0.0, jnp.float32)
        _vfill(d2a, G, 0.0, jnp.float32)
        tfv = tf_v[pl.ds(0, L)] != 0

        def step(j, _):
            sl = pl.ds(j * L, L)
            idx = idx_v[sl]
            v0 = l0_v[sl]
            m0g = plsc.load_gather(m0f_v, [idx])
            s0_v[sl] = jnp.where(tfv, g0_v[sl],
                                 (v0 >= m0g).astype(jnp.float32))
            plsc.addupdate_scatter(d0a, [idx], jnp.exp(v0 - m0g))
            idx2 = plsc.load_gather(bv_v, [idx])
            v1 = l1_v[sl]
            m2g = plsc.load_gather(m2f_v, [idx2])
            s1_v[sl] = jnp.where(tfv, g1_v[sl],
                                 (v1 >= m2g).astype(jnp.float32))
            plsc.addupdate_scatter(d2a, [idx2], jnp.exp(v1 - m2g))
            return 0

        lax.fori_loop(0, CH // L, step, 0)
        pltpu.sync_copy(s0_v, s0_h.at[pl.ds(base, CH)])
        pltpu.sync_copy(s1_v, s1_h.at[pl.ds(base, CH)])
        pltpu.sync_copy(d0a, d0p_h.at[wid])
        pltpu.sync_copy(d2a, d2p_h.at[wid])

    return k(e_idx, l0, l1, g0, g1, batch_vec, m0f, m2f, tf16)


def _c2_body(d0p_ref, d2p_ref, m0f_ref, g0f_ref, m2f_ref, g2f_ref,
             dp0_ref, dp1_ref, np_ref, out_ref):
    d0f = jnp.sum(d0p_ref[...], axis=0, keepdims=True)
    d2f = jnp.sum(d2p_ref[...], axis=0, keepdims=True)
    dot0 = jnp.sum(dp0_ref[...])
    dot1 = jnp.sum(dp1_ref[...])
    npv = np_ref[...]
    ii = lax.broadcasted_iota(jnp.int32, (1, G), 1)
    loss_node = jnp.sum(jnp.where(ii == 0, npv, 0.0))
    n0count = jnp.sum(jnp.where(ii == 1, npv, 0.0))
    g0f = g0f_ref[...]
    m0f = m0f_ref[...]
    g2f = g2f_ref[...]
    m2f = m2f_ref[...]
    loss_a = (-dot0 + jnp.sum(g0f * m0f) +
              jnp.sum(g0f * jnp.log(d0f + 1e-20))) / N_NODES
    loss_b = n0count * (-dot1 + jnp.sum(g2f * m2f) +
                        jnp.sum(g2f * jnp.log(d2f + 1e-20))) / G
    out_ref[...] = jnp.full((1, 1), loss_node + loss_a + loss_b, jnp.float32)


def _c2(d0p, d2p, m0f, g0f, m2f, g2f, dp0, dp1, npart):
    return pl.pallas_call(
        _c2_body,
        grid=(1,),
        in_specs=[
            pl.BlockSpec((NW, S0P), lambda i: (0, 0)),
            pl.BlockSpec((NW, G), lambda i: (0, 0)),
            pl.BlockSpec((1, S0P), lambda i: (0, 0)),
            pl.BlockSpec((1, S0P), lambda i: (0, 0)),
            pl.BlockSpec((1, G), lambda i: (0, 0)),
            pl.BlockSpec((1, G), lambda i: (0, 0)),
            pl.BlockSpec((NW, L), lambda i: (0, 0)),
            pl.BlockSpec((NW, L), lambda i: (0, 0)),
            pl.BlockSpec((1, G), lambda i: (0, 0)),
        ],
        out_specs=pl.BlockSpec((1, 1), lambda i: (0, 0)),
        out_shape=jax.ShapeDtypeStruct((1, 1), jnp.float32),
    )(d0p, d2p, m0f, g0f, m2f, g2f, dp0, dp1, npart)


def kernel(node_fts, edge_fts, node_hints, edge_hints, W_node, b_node, W_edge,
           b_edge, batch_vec, edge_index, processor_step, training_step,
           teacher_force):
    step = jnp.asarray(processor_step, jnp.int32)
    tf_i = jnp.asarray(teacher_force, jnp.int32).reshape(1)
    tf16 = jnp.broadcast_to(tf_i, (L,))
    batch_vec = batch_vec.astype(jnp.int32)

    g0 = lax.dynamic_slice(edge_hints, (0, step, 0),
                           (N_EDGES, 1, 1)).reshape(N_EDGES)
    g1 = lax.dynamic_slice(edge_hints, (0, step, 1),
                           (N_EDGES, 1, 1)).reshape(N_EDGES)
    gtn = [lax.dynamic_slice(node_hints, (0, step, k),
                             (N_NODES, 1, 1)).reshape(N_NODES, 1)
           for k in range(3)]

    l0, l1, e_idx = _edge_logits(edge_fts, W_edge, b_edge,
                                 edge_index.astype(jnp.int32))

    m0p, g0p, m2p, g2p, dp0, dp1 = _k1_partials(
        e_idx, l0, g0, l1, g1, batch_vec)
    states_n, npart = _c1a(node_fts, W_node, b_node, gtn[0], gtn[1], gtn[2],
                           batch_vec, tf_i)
    m0f, g0f, m2f, g2f = _c1b(m0p, g0p, m2p, g2p)
    s0, s1, d0p, d2p = _k3_denoms_states(
        e_idx, l0, l1, g0, g1, batch_vec, m0f.reshape(S0P), m2f.reshape(G),
        tf16)
    loss11 = _c2(d0p, d2p, m0f, g0f, m2f, g2f, dp0, dp1, npart)

    loss = loss11[0, 0]
    states_e = jnp.stack([s0, s1], axis=-1)
    return (states_n, states_e, loss)


# speedup vs baseline: 55.4820x; 1.0443x over previous
import functools

import jax
import jax.numpy as jnp
from jax import lax
from jax.experimental import pallas as pl
from jax.experimental.pallas import tpu as pltpu
from jax.experimental.pallas import tpu_sc as plsc

N_NODES = 10000
N_EDGES = 320000
H = 128
G = 128
EBLK = 16384
S0P = 10112
NW = 32
CH = N_EDGES // NW
L = 16
NEG = -3.4e38

_SC_PARAMS = pltpu.CompilerParams(needs_layout_passes=False)


def _sc_mesh():
    return plsc.VectorSubcoreMesh(
        core_axis_name="c", subcore_axis_name="s", num_cores=2, num_subcores=16)


def _a_body(fts_ref, w_ref, b_ref, ei_ref, l0_ref, l1_ref, idx_ref):
    lg = lax.dot_general(w_ref[...], fts_ref[...],
                         (((1,), (1,)), ((), ())))
    lg = lg + b_ref[...]
    l0_ref[...] = lg[0]
    l1_ref[...] = lg[1]
    idx_ref[...] = ei_ref[0]


def _edge_logits(edge_fts, W_edge, b_edge, edge_index):
    return pl.pallas_call(
        _a_body,
        grid=((N_EDGES + EBLK - 1) // EBLK,),
        in_specs=[
            pl.BlockSpec((EBLK, H), lambda i: (i, 0)),
            pl.BlockSpec((2, H), lambda i: (0, 0)),
            pl.BlockSpec((2, 1), lambda i: (0, 0)),
            pl.BlockSpec((2, EBLK), lambda i: (0, i)),
        ],
        out_specs=[
            pl.BlockSpec((EBLK,), lambda i: (i,)),
            pl.BlockSpec((EBLK,), lambda i: (i,)),
            pl.BlockSpec((EBLK,), lambda i: (i,)),
        ],
        out_shape=[
            jax.ShapeDtypeStruct((N_EDGES,), jnp.float32),
            jax.ShapeDtypeStruct((N_EDGES,), jnp.float32),
            jax.ShapeDtypeStruct((N_EDGES,), jnp.int32),
        ],
    )(edge_fts, W_edge, b_edge.reshape(2, 1), edge_index)


def _scatter_max16(acc, idx, val):
    def cond(act):
        return jnp.any(act)

    def body(act):
        cur = plsc.load_gather(acc, [idx])
        need = jnp.logical_and(act, val > cur)
        plsc.store_scatter(acc, [idx], val, mask=need)
        cur2 = plsc.load_gather(acc, [idx])
        return jnp.logical_and(need, val > cur2)

    act0 = val > plsc.load_gather(acc, [idx])
    lax.while_loop(cond, body, act0)


def _scatter_max16_pair(acc_a, idx_a, val_a, acc_b, idx_b, val_b):
    def cond(st):
        aa, ab = st
        return jnp.any(jnp.logical_or(aa, ab))

    def body(st):
        aa, ab = st
        cura = plsc.load_gather(acc_a, [idx_a])
        needa = jnp.logical_and(aa, val_a > cura)
        plsc.store_scatter(acc_a, [idx_a], val_a, mask=needa)
        curb = plsc.load_gather(acc_b, [idx_b])
        needb = jnp.logical_and(ab, val_b > curb)
        plsc.store_scatter(acc_b, [idx_b], val_b, mask=needb)
        cura2 = plsc.load_gather(acc_a, [idx_a])
        curb2 = plsc.load_gather(acc_b, [idx_b])
        return (jnp.logical_and(needa, val_a > cura2),
                jnp.logical_and(needb, val_b > curb2))

    aa0 = val_a > plsc.load_gather(acc_a, [idx_a])
    ab0 = val_b > plsc.load_gather(acc_b, [idx_b])
    lax.while_loop(cond, body, (aa0, ab0))


def _vfill(ref, n, value, dtype):
    def body(i, _):
        ref[pl.ds(i * L, L)] = jnp.full((L,), value, dtype)
        return 0

    lax.fori_loop(0, n // L, body, 0)


def _k1_partials(e_idx, l0, g0, l1, g1, batch_vec):
    @functools.partial(
        pl.kernel,
        out_type=(
            jax.ShapeDtypeStruct((NW, S0P), jnp.float32),
            jax.ShapeDtypeStruct((NW, S0P), jnp.float32),
            jax.ShapeDtypeStruct((NW, G), jnp.float32),
            jax.ShapeDtypeStruct((NW, G), jnp.float32),
            jax.ShapeDtypeStruct((NW, L), jnp.float32),
            jax.ShapeDtypeStruct((NW, L), jnp.float32),
        ),
        mesh=_sc_mesh(),
        compiler_params=_SC_PARAMS,
        scratch_types=[
            pltpu.VMEM((CH,), jnp.int32),
            pltpu.VMEM((CH,), jnp.float32),
            pltpu.VMEM((CH,), jnp.float32),
            pltpu.VMEM((CH,), jnp.float32),
            pltpu.VMEM((CH,), jnp.float32),
            pltpu.VMEM((N_NODES,), jnp.int32),
            pltpu.VMEM((S0P,), jnp.float32),
            pltpu.VMEM((S0P,), jnp.float32),
            pltpu.VMEM((G,), jnp.float32),
            pltpu.VMEM((G,), jnp.float32),
            pltpu.VMEM((L,), jnp.float32),
        ],
    )
    def k(idx_h, l0_h, g0_h, l1_h, g1_h, bv_h,
          m0p_h, g0p_h, m2p_h, g2p_h, dp0_h, dp1_h,
          idx_v, l0_v, g0_v, l1_v, g1_v, bv_v, m0a, g0a, m2a, g2a, dt_v):
        wid = lax.axis_index("s") * 2 + lax.axis_index("c")
        base = wid * CH
        pltpu.sync_copy(idx_h.at[pl.ds(base, CH)], idx_v)
        pltpu.sync_copy(l0_h.at[pl.ds(base, CH)], l0_v)
        pltpu.sync_copy(g0_h.at[pl.ds(base, CH)], g0_v)
        pltpu.sync_copy(l1_h.at[pl.ds(base, CH)], l1_v)
        pltpu.sync_copy(g1_h.at[pl.ds(base, CH)], g1_v)
        pltpu.sync_copy(bv_h, bv_v)
        _vfill(m0a, S0P, NEG, jnp.float32)
        _vfill(g0a, S0P, 0.0, jnp.float32)
        _vfill(m2a, G, NEG, jnp.float32)
        _vfill(g2a, G, 0.0, jnp.float32)

        def step(j, carry):
            dv0, dv1 = carry
            sl = pl.ds(j * L, L)
            idx = idx_v[sl]
            lv0 = l0_v[sl]
            gv0 = g0_v[sl]
            idx2 = plsc.load_gather(bv_v, [idx])
            lv1 = l1_v[sl]
            gv1 = g1_v[sl]
            plsc.addupdate_scatter(g0a, [idx], gv0)
            plsc.addupdate_scatter(g2a, [idx2], gv1)
            _scatter_max16_pair(m0a, idx, lv0, m2a, idx2, lv1)
            return (dv0 + gv0 * lv0, dv1 + gv1 * lv1)

        zero = jnp.zeros((L,), jnp.float32)
        dv0, dv1 = lax.fori_loop(0, CH // L, step, (zero, zero))
        pltpu.sync_copy(m0a, m0p_h.at[wid])
        pltpu.sync_copy(g0a, g0p_h.at[wid])
        pltpu.sync_copy(m2a, m2p_h.at[wid])
        pltpu.sync_copy(g2a, g2p_h.at[wid])
        dt_v[pl.ds(0, L)] = dv0
        pltpu.sync_copy(dt_v, dp0_h.at[wid])
        dt_v[pl.ds(0, L)] = dv1
        pltpu.sync_copy(dt_v, dp1_h.at[wid])

    return k(e_idx, l0, g0, l1, g1, batch_vec)


_CB = 2500


def _c1a_chunk(c, nf_ref, w_ref, b_ref, g0_ref, g1_ref, g2_ref, bv_ref):
    sl = pl.ds(c * _CB, _CB)
    x = nf_ref[sl, :]
    logits = lax.dot_general(x, w_ref[...],
                             (((1,), (1,)), ((), ())))
    logits = logits + b_ref[...]
    gt = jnp.concatenate([g0_ref[sl, :], g1_ref[sl, :], g2_ref[sl, :]],
                         axis=1)
    bv = bv_ref[sl, :]
    onehot = bv == lax.broadcasted_iota(jnp.int32, (_CB, G), 1)
    return logits, gt, onehot


def _c1a_body(tf_ref, nf_ref, w_ref, b_ref, g0_ref, g1_ref, g2_ref, bv_ref,
              states_ref, np_ref):
    def ph1(c, carry):
        m_n, gseg, dotn, bce1, bce2, n0c = carry
        logits, gt, onehot = _c1a_chunk(c, nf_ref, w_ref, b_ref, g0_ref,
                                        g1_ref, g2_ref, bv_ref)
        l0 = logits[:, 0:1]
        g0 = gt[:, 0:1]
        m_n = jnp.maximum(m_n, jnp.max(jnp.where(onehot, l0, NEG), axis=0,
                                       keepdims=True))
        gseg = gseg + jnp.sum(jnp.where(onehot, g0, 0.0), axis=0,
                              keepdims=True)
        dotn = dotn + jnp.sum(g0 * l0)
        l1 = logits[:, 1:2]
        g1 = gt[:, 1:2]
        bce1 = bce1 + jnp.sum(jnp.maximum(l1, 0.0) - l1 * g1 +
                              jnp.log1p(jnp.exp(-jnp.abs(l1))))
        l2 = logits[:, 2:3]
        g2 = gt[:, 2:3]
        bce2 = bce2 + jnp.sum(jnp.maximum(l2, 0.0) - l2 * g2 +
                              jnp.log1p(jnp.exp(-jnp.abs(l2))))
        n0c = n0c + jnp.sum(jnp.where(onehot[:, 0:1], 1.0, 0.0))
        return m_n, gseg, dotn, bce1, bce2, n0c

    init = (jnp.full((1, G), NEG, jnp.float32),
            jnp.zeros((1, G), jnp.float32),
            jnp.float32(0.0), jnp.float32(0.0), jnp.float32(0.0),
            jnp.float32(0.0))
    m_n, gseg, dotn, bce1, bce2, n0c = lax.fori_loop(
        0, N_NODES // _CB, ph1, init)

    def ph2(c, denom):
        logits, gt, onehot = _c1a_chunk(c, nf_ref, w_ref, b_ref, g0_ref,
                                        g1_ref, g2_ref, bv_ref)
        l0 = logits[:, 0:1]
        m_gath = jnp.sum(jnp.where(onehot, m_n, 0.0), axis=1, keepdims=True)
        denom = denom + jnp.sum(jnp.where(onehot, jnp.exp(l0 - m_gath), 0.0),
                                axis=0, keepdims=True)
        preds = jnp.concatenate(
            [(l0 >= m_gath).astype(jnp.float32),
             (logits[:, 1:2] > 0.0).astype(jnp.float32),
             (logits[:, 2:3] > 0.0).astype(jnp.float32)], axis=1)
        states_ref[pl.ds(c * _CB, _CB), :] = jnp.where(tf_ref[0] != 0, gt,
                                                       preds)
        return denom

    denom = lax.fori_loop(0, N_NODES // _CB, ph2,
                          jnp.zeros((1, G), jnp.float32))

    loss_n0 = (-dotn + jnp.sum(gseg * m_n) +
               jnp.sum(gseg * jnp.log(denom + 1e-20))) / G
    loss_node = loss_n0 + bce1 / N_NODES + bce2 / N_NODES
    ii = lax.broadcasted_iota(jnp.int32, (1, G), 1)
    np_ref[...] = jnp.where(ii == 0, loss_node,
                            jnp.where(ii == 1, n0c, 0.0))


def _c1a(node_fts, W_node, b_node, gtn0, gtn1, gtn2, batch_vec, tf_i):
    return pl.pallas_call(
        _c1a_body,
        grid=(1,),
        in_specs=[
            pl.BlockSpec(memory_space=pltpu.SMEM),
            pl.BlockSpec((N_NODES, H), lambda i: (0, 0)),
            pl.BlockSpec((3, H), lambda i: (0, 0)),
            pl.BlockSpec((1, 3), lambda i: (0, 0)),
            pl.BlockSpec((N_NODES, 1), lambda i: (0, 0)),
            pl.BlockSpec((N_NODES, 1), lambda i: (0, 0)),
            pl.BlockSpec((N_NODES, 1), lambda i: (0, 0)),
            pl.BlockSpec((N_NODES, 1), lambda i: (0, 0)),
        ],
        out_specs=[
            pl.BlockSpec((N_NODES, 3), lambda i: (0, 0)),
            pl.BlockSpec((1, G), lambda i: (0, 0)),
        ],
        out_shape=[
            jax.ShapeDtypeStruct((N_NODES, 3), jnp.float32),
            jax.ShapeDtypeStruct((1, G), jnp.float32),
        ],
    )(tf_i, node_fts, W_node, b_node.reshape(1, 3), gtn0, gtn1, gtn2,
      batch_vec.reshape(N_NODES, 1))


def _c1b_body(m0p_ref, g0p_ref, m2p_ref, g2p_ref,
              m0f_ref, g0f_ref, m2f_ref, g2f_ref):
    m0f_ref[...] = jnp.max(m0p_ref[...], axis=0, keepdims=True)
    g0f_ref[...] = jnp.sum(g0p_ref[...], axis=0, keepdims=True)
    m2f_ref[...] = jnp.max(m2p_ref[...], axis=0, keepdims=True)
    g2f_ref[...] = jnp.sum(g2p_ref[...], axis=0, keepdims=True)


def _c1b(m0p, g0p, m2p, g2p):
    return pl.pallas_call(
        _c1b_body,
        grid=(1,),
        in_specs=[
            pl.BlockSpec((NW, S0P), lambda i: (0, 0)),
            pl.BlockSpec((NW, S0P), lambda i: (0, 0)),
            pl.BlockSpec((NW, G), lambda i: (0, 0)),
            pl.BlockSpec((NW, G), lambda i: (0, 0)),
        ],
        out_specs=[
            pl.BlockSpec((1, S0P), lambda i: (0, 0)),
            pl.BlockSpec((1, S0P), lambda i: (0, 0)),
            pl.BlockSpec((1, G), lambda i: (0, 0)),
            pl.BlockSpec((1, G), lambda i: (0, 0)),
        ],
        out_shape=[
            jax.ShapeDtypeStruct((1, S0P), jnp.float32),
            jax.ShapeDtypeStruct((1, S0P), jnp.float32),
            jax.ShapeDtypeStruct((1, G), jnp.float32),
            jax.ShapeDtypeStruct((1, G), jnp.float32),
        ],
    )(m0p, g0p, m2p, g2p)


def _k3_denoms_states(e_idx, l0, l1, g0, g1, batch_vec, m0f, m2f, tf16):
    @functools.partial(
        pl.kernel,
        out_type=(
            jax.ShapeDtypeStruct((N_EDGES,), jnp.float32),
            jax.ShapeDtypeStruct((N_EDGES,), jnp.float32),
            jax.ShapeDtypeStruct((NW, S0P), jnp.float32),
            jax.ShapeDtypeStruct((NW, G), jnp.float32),
        ),
        mesh=_sc_mesh(),
        compiler_params=_SC_PARAMS,
        scratch_types=[
            pltpu.VMEM((CH,), jnp.int32),
            pltpu.VMEM((CH,), jnp.float32),
            pltpu.VMEM((CH,), jnp.float32),
            pltpu.VMEM((CH,), jnp.float32),
            pltpu.VMEM((CH,), jnp.float32),
            pltpu.VMEM((N_NODES,), jnp.int32),
            pltpu.VMEM((S0P,), jnp.float32),
            pltpu.VMEM((G,), jnp.float32),
            pltpu.VMEM((CH,), jnp.float32),
            pltpu.VMEM((CH,), jnp.float32),
            pltpu.VMEM((S0P,), jnp.float32),
            pltpu.VMEM((G,), jnp.float32),
            pltpu.VMEM((L,), jnp.int32),
        ],
    )
    def k(idx_h, l0_h, l1_h, g0_h, g1_h, bv_h, m0f_h, m2f_h, tf_h,
          s0_h, s1_h, d0p_h, d2p_h,
          idx_v, l0_v, l1_v, g0_v, g1_v, bv_v, m0f_v, m2f_v, s0_v, s1_v,
          d0a, d2a, tf_v):
        wid = lax.axis_index("s") * 2 + lax.axis_index("c")
        base = wid * CH
        pltpu.sync_copy(idx_h.at[pl.ds(base, CH)], idx_v)
        pltpu.sync_copy(l0_h.at[pl.ds(base, CH)], l0_v)
        pltpu.sync_copy(l1_h.at[pl.ds(base, CH)], l1_v)
        pltpu.sync_copy(g0_h.at[pl.ds(base, CH)], g0_v)
        pltpu.sync_copy(g1_h.at[pl.ds(base, CH)], g1_v)
        pltpu.sync_copy(bv_h, bv_v)
        pltpu.sync_copy(m0f_h, m0f_v)
        pltpu.sync_copy(m2f_h, m2f_v)
        pltpu.sync_copy(tf_h, tf_v)
        _vfill(d0a, S0P, 0.0, jnp.float32)
        _vfill(d2a, G, 0.0, jnp.float32)
        tfv = tf_v[pl.ds(0, L)] != 0

        def step(j, _):
            sl = pl.ds(j * L, L)
            idx = idx_v[sl]
            v0 = l0_v[sl]
            m0g = plsc.load_gather(m0f_v, [idx])
            s0_v[sl] = jnp.where(tfv, g0_v[sl],
                                 (v0 >= m0g).astype(jnp.float32))
            plsc.addupdate_scatter(d0a, [idx], jnp.exp(v0 - m0g))
            idx2 = plsc.load_gather(bv_v, [idx])
            v1 = l1_v[sl]
            m2g = plsc.load_gather(m2f_v, [idx2])
            s1_v[sl] = jnp.where(tfv, g1_v[sl],
                                 (v1 >= m2g).astype(jnp.float32))
            plsc.addupdate_scatter(d2a, [idx2], jnp.exp(v1 - m2g))
            return 0

        lax.fori_loop(0, CH // L, step, 0)
        pltpu.sync_copy(s0_v, s0_h.at[pl.ds(base, CH)])
        pltpu.sync_copy(s1_v, s1_h.at[pl.ds(base, CH)])
        pltpu.sync_copy(d0a, d0p_h.at[wid])
        pltpu.sync_copy(d2a, d2p_h.at[wid])

    return k(e_idx, l0, l1, g0, g1, batch_vec, m0f, m2f, tf16)


def _c2_body(d0p_ref, d2p_ref, m0f_ref, g0f_ref, m2f_ref, g2f_ref,
             dp0_ref, dp1_ref, np_ref, out_ref):
    d0f = jnp.sum(d0p_ref[...], axis=0, keepdims=True)
    d2f = jnp.sum(d2p_ref[...], axis=0, keepdims=True)
    dot0 = jnp.sum(dp0_ref[...])
    dot1 = jnp.sum(dp1_ref[...])
    npv = np_ref[...]
    ii = lax.broadcasted_iota(jnp.int32, (1, G), 1)
    loss_node = jnp.sum(jnp.where(ii == 0, npv, 0.0))
    n0count = jnp.sum(jnp.where(ii == 1, npv, 0.0))
    g0f = g0f_ref[...]
    m0f = m0f_ref[...]
    g2f = g2f_ref[...]
    m2f = m2f_ref[...]
    loss_a = (-dot0 + jnp.sum(g0f * m0f) +
              jnp.sum(g0f * jnp.log(d0f + 1e-20))) / N_NODES
    loss_b = n0count * (-dot1 + jnp.sum(g2f * m2f) +
                        jnp.sum(g2f * jnp.log(d2f + 1e-20))) / G
    out_ref[...] = jnp.full((1, 1), loss_node + loss_a + loss_b, jnp.float32)


def _c2(d0p, d2p, m0f, g0f, m2f, g2f, dp0, dp1, npart):
    return pl.pallas_call(
        _c2_body,
        grid=(1,),
        in_specs=[
            pl.BlockSpec((NW, S0P), lambda i: (0, 0)),
            pl.BlockSpec((NW, G), lambda i: (0, 0)),
            pl.BlockSpec((1, S0P), lambda i: (0, 0)),
            pl.BlockSpec((1, S0P), lambda i: (0, 0)),
            pl.BlockSpec((1, G), lambda i: (0, 0)),
            pl.BlockSpec((1, G), lambda i: (0, 0)),
            pl.BlockSpec((NW, L), lambda i: (0, 0)),
            pl.BlockSpec((NW, L), lambda i: (0, 0)),
            pl.BlockSpec((1, G), lambda i: (0, 0)),
        ],
        out_specs=pl.BlockSpec((1, 1), lambda i: (0, 0)),
        out_shape=jax.ShapeDtypeStruct((1, 1), jnp.float32),
    )(d0p, d2p, m0f, g0f, m2f, g2f, dp0, dp1, npart)


def kernel(node_fts, edge_fts, node_hints, edge_hints, W_node, b_node, W_edge,
           b_edge, batch_vec, edge_index, processor_step, training_step,
           teacher_force):
    step = jnp.asarray(processor_step, jnp.int32)
    tf_i = jnp.asarray(teacher_force, jnp.int32).reshape(1)
    tf16 = jnp.broadcast_to(tf_i, (L,))
    batch_vec = batch_vec.astype(jnp.int32)

    g0 = lax.dynamic_slice(edge_hints, (0, step, 0),
                           (N_EDGES, 1, 1)).reshape(N_EDGES)
    g1 = lax.dynamic_slice(edge_hints, (0, step, 1),
                           (N_EDGES, 1, 1)).reshape(N_EDGES)
    gtn = [lax.dynamic_slice(node_hints, (0, step, k),
                             (N_NODES, 1, 1)).reshape(N_NODES, 1)
           for k in range(3)]

    l0, l1, e_idx = _edge_logits(edge_fts, W_edge, b_edge,
                                 edge_index.astype(jnp.int32))

    m0p, g0p, m2p, g2p, dp0, dp1 = _k1_partials(
        e_idx, l0, g0, l1, g1, batch_vec)
    states_n, npart = _c1a(node_fts, W_node, b_node, gtn[0], gtn[1], gtn[2],
                           batch_vec, tf_i)
    m0f, g0f, m2f, g2f = _c1b(m0p, g0p, m2p, g2p)
    s0, s1, d0p, d2p = _k3_denoms_states(
        e_idx, l0, l1, g0, g1, batch_vec, m0f.reshape(S0P), m2f.reshape(G),
        tf16)
    loss11 = _c2(d0p, d2p, m0f, g0f, m2f, g2f, dp0, dp1, npart)

    loss = loss11[0, 0]
    states_e = jnp.stack([s0, s1], axis=-1)
    return (states_n, states_e, loss)


# speedup vs baseline: 61.5449x; 1.1093x over previous
import functools

import jax
import jax.numpy as jnp
from jax import lax
from jax.experimental import pallas as pl
from jax.experimental.pallas import tpu as pltpu
from jax.experimental.pallas import tpu_sc as plsc

N_NODES = 10000
N_EDGES = 320000
H = 128
G = 128
EBLK = 16384
S0P = 10112
NW = 32
CH = N_EDGES // NW
L = 16
NEG = -3.4e38

_SC_PARAMS = pltpu.CompilerParams(needs_layout_passes=False)


def _sc_mesh():
    return plsc.VectorSubcoreMesh(
        core_axis_name="c", subcore_axis_name="s", num_cores=2, num_subcores=16)


def _a_body(fts_ref, w_ref, b_ref, ei_ref, l0_ref, l1_ref, idx_ref):
    lg = lax.dot_general(w_ref[...], fts_ref[...],
                         (((1,), (1,)), ((), ())))
    lg = lg + b_ref[...]
    l0_ref[...] = lg[0]
    l1_ref[...] = lg[1]
    idx_ref[...] = ei_ref[0]


def _edge_logits(edge_fts, W_edge, b_edge, edge_index):
    return pl.pallas_call(
        _a_body,
        grid=((N_EDGES + EBLK - 1) // EBLK,),
        in_specs=[
            pl.BlockSpec((EBLK, H), lambda i: (i, 0)),
            pl.BlockSpec((2, H), lambda i: (0, 0)),
            pl.BlockSpec((2, 1), lambda i: (0, 0)),
            pl.BlockSpec((2, EBLK), lambda i: (0, i)),
        ],
        out_specs=[
            pl.BlockSpec((EBLK,), lambda i: (i,)),
            pl.BlockSpec((EBLK,), lambda i: (i,)),
            pl.BlockSpec((EBLK,), lambda i: (i,)),
        ],
        out_shape=[
            jax.ShapeDtypeStruct((N_EDGES,), jnp.float32),
            jax.ShapeDtypeStruct((N_EDGES,), jnp.float32),
            jax.ShapeDtypeStruct((N_EDGES,), jnp.int32),
        ],
    )(edge_fts, W_edge, b_edge.reshape(2, 1), edge_index)


def _scatter_max16(acc, idx, val):
    def cond(act):
        return jnp.any(act)

    def body(act):
        cur = plsc.load_gather(acc, [idx])
        need = jnp.logical_and(act, val > cur)
        plsc.store_scatter(acc, [idx], val, mask=need)
        cur2 = plsc.load_gather(acc, [idx])
        return jnp.logical_and(need, val > cur2)

    act0 = val > plsc.load_gather(acc, [idx])
    lax.while_loop(cond, body, act0)


def _scatter_max16_pair(acc_a, idx_a, val_a, acc_b, idx_b, val_b):
    def cond(st):
        aa, ab = st
        return jnp.any(jnp.logical_or(aa, ab))

    def body(st):
        aa, ab = st
        cura = plsc.load_gather(acc_a, [idx_a])
        needa = jnp.logical_and(aa, val_a > cura)
        plsc.store_scatter(acc_a, [idx_a], val_a, mask=needa)
        curb = plsc.load_gather(acc_b, [idx_b])
        needb = jnp.logical_and(ab, val_b > curb)
        plsc.store_scatter(acc_b, [idx_b], val_b, mask=needb)
        cura2 = plsc.load_gather(acc_a, [idx_a])
        curb2 = plsc.load_gather(acc_b, [idx_b])
        return (jnp.logical_and(needa, val_a > cura2),
                jnp.logical_and(needb, val_b > curb2))

    aa0 = val_a > plsc.load_gather(acc_a, [idx_a])
    ab0 = val_b > plsc.load_gather(acc_b, [idx_b])
    lax.while_loop(cond, body, (aa0, ab0))


def _vfill(ref, n, value, dtype):
    def body(i, _):
        ref[pl.ds(i * L, L)] = jnp.full((L,), value, dtype)
        return 0

    lax.fori_loop(0, n // L, body, 0)


def _k1_partials(e_idx, l0, l1, batch_vec):
    @functools.partial(
        pl.kernel,
        out_type=(
            jax.ShapeDtypeStruct((NW, S0P), jnp.float32),
            jax.ShapeDtypeStruct((NW, G), jnp.float32),
        ),
        mesh=_sc_mesh(),
        compiler_params=_SC_PARAMS,
        scratch_types=[
            pltpu.VMEM((CH,), jnp.int32),
            pltpu.VMEM((CH,), jnp.float32),
            pltpu.VMEM((CH,), jnp.float32),
            pltpu.VMEM((N_NODES,), jnp.int32),
            pltpu.VMEM((S0P,), jnp.float32),
            pltpu.VMEM((G,), jnp.float32),
        ],
    )
    def k(idx_h, l0_h, l1_h, bv_h, m0p_h, m2p_h,
          idx_v, l0_v, l1_v, bv_v, m0a, m2a):
        wid = lax.axis_index("s") * 2 + lax.axis_index("c")
        base = wid * CH
        pltpu.sync_copy(idx_h.at[pl.ds(base, CH)], idx_v)
        pltpu.sync_copy(l0_h.at[pl.ds(base, CH)], l0_v)
        pltpu.sync_copy(l1_h.at[pl.ds(base, CH)], l1_v)
        pltpu.sync_copy(bv_h, bv_v)
        _vfill(m0a, S0P, NEG, jnp.float32)
        _vfill(m2a, G, NEG, jnp.float32)

        def step(j, _):
            sl = pl.ds(j * L, L)
            idx = idx_v[sl]
            lv0 = l0_v[sl]
            idx2 = plsc.load_gather(bv_v, [idx])
            lv1 = l1_v[sl]
            _scatter_max16_pair(m0a, idx, lv0, m2a, idx2, lv1)
            return 0

        lax.fori_loop(0, CH // L, step, 0)
        pltpu.sync_copy(m0a, m0p_h.at[wid])
        pltpu.sync_copy(m2a, m2p_h.at[wid])

    return k(e_idx, l0, l1, batch_vec)


_CB = 2500


def _c1a_chunk(c, nf_ref, w_ref, b_ref, g0_ref, g1_ref, g2_ref, bv_ref):
    sl = pl.ds(c * _CB, _CB)
    x = nf_ref[sl, :]
    logits = lax.dot_general(x, w_ref[...],
                             (((1,), (1,)), ((), ())))
    logits = logits + b_ref[...]
    gt = jnp.concatenate([g0_ref[sl, :], g1_ref[sl, :], g2_ref[sl, :]],
                         axis=1)
    bv = bv_ref[sl, :]
    onehot = bv == lax.broadcasted_iota(jnp.int32, (_CB, G), 1)
    return logits, gt, onehot


def _c1a_body(tf_ref, nf_ref, w_ref, b_ref, g0_ref, g1_ref, g2_ref, bv_ref,
              states_ref, np_ref):
    def ph1(c, carry):
        m_n, gseg, dotn, bce1, bce2, n0c = carry
        logits, gt, onehot = _c1a_chunk(c, nf_ref, w_ref, b_ref, g0_ref,
                                        g1_ref, g2_ref, bv_ref)
        l0 = logits[:, 0:1]
        g0 = gt[:, 0:1]
        m_n = jnp.maximum(m_n, jnp.max(jnp.where(onehot, l0, NEG), axis=0,
                                       keepdims=True))
        gseg = gseg + jnp.sum(jnp.where(onehot, g0, 0.0), axis=0,
                              keepdims=True)
        dotn = dotn + jnp.sum(g0 * l0)
        l1 = logits[:, 1:2]
        g1 = gt[:, 1:2]
        bce1 = bce1 + jnp.sum(jnp.maximum(l1, 0.0) - l1 * g1 +
                              jnp.log1p(jnp.exp(-jnp.abs(l1))))
        l2 = logits[:, 2:3]
        g2 = gt[:, 2:3]
        bce2 = bce2 + jnp.sum(jnp.maximum(l2, 0.0) - l2 * g2 +
                              jnp.log1p(jnp.exp(-jnp.abs(l2))))
        n0c = n0c + jnp.sum(jnp.where(onehot[:, 0:1], 1.0, 0.0))
        return m_n, gseg, dotn, bce1, bce2, n0c

    init = (jnp.full((1, G), NEG, jnp.float32),
            jnp.zeros((1, G), jnp.float32),
            jnp.float32(0.0), jnp.float32(0.0), jnp.float32(0.0),
            jnp.float32(0.0))
    m_n, gseg, dotn, bce1, bce2, n0c = lax.fori_loop(
        0, N_NODES // _CB, ph1, init)

    def ph2(c, denom):
        logits, gt, onehot = _c1a_chunk(c, nf_ref, w_ref, b_ref, g0_ref,
                                        g1_ref, g2_ref, bv_ref)
        l0 = logits[:, 0:1]
        m_gath = jnp.sum(jnp.where(onehot, m_n, 0.0), axis=1, keepdims=True)
        denom = denom + jnp.sum(jnp.where(onehot, jnp.exp(l0 - m_gath), 0.0),
                                axis=0, keepdims=True)
        preds = jnp.concatenate(
            [(l0 >= m_gath).astype(jnp.float32),
             (logits[:, 1:2] > 0.0).astype(jnp.float32),
             (logits[:, 2:3] > 0.0).astype(jnp.float32)], axis=1)
        states_ref[pl.ds(c * _CB, _CB), :] = jnp.where(tf_ref[0] != 0, gt,
                                                       preds)
        return denom

    denom = lax.fori_loop(0, N_NODES // _CB, ph2,
                          jnp.zeros((1, G), jnp.float32))

    loss_n0 = (-dotn + jnp.sum(gseg * m_n) +
               jnp.sum(gseg * jnp.log(denom + 1e-20))) / G
    loss_node = loss_n0 + bce1 / N_NODES + bce2 / N_NODES
    ii = lax.broadcasted_iota(jnp.int32, (1, G), 1)
    np_ref[...] = jnp.where(ii == 0, loss_node,
                            jnp.where(ii == 1, n0c, 0.0))


def _c1a(node_fts, W_node, b_node, gtn0, gtn1, gtn2, batch_vec, tf_i):
    return pl.pallas_call(
        _c1a_body,
        grid=(1,),
        in_specs=[
            pl.BlockSpec(memory_space=pltpu.SMEM),
            pl.BlockSpec((N_NODES, H), lambda i: (0, 0)),
            pl.BlockSpec((3, H), lambda i: (0, 0)),
            pl.BlockSpec((1, 3), lambda i: (0, 0)),
            pl.BlockSpec((N_NODES, 1), lambda i: (0, 0)),
            pl.BlockSpec((N_NODES, 1), lambda i: (0, 0)),
            pl.BlockSpec((N_NODES, 1), lambda i: (0, 0)),
            pl.BlockSpec((N_NODES, 1), lambda i: (0, 0)),
        ],
        out_specs=[
            pl.BlockSpec((N_NODES, 3), lambda i: (0, 0)),
            pl.BlockSpec((1, G), lambda i: (0, 0)),
        ],
        out_shape=[
            jax.ShapeDtypeStruct((N_NODES, 3), jnp.float32),
            jax.ShapeDtypeStruct((1, G), jnp.float32),
        ],
    )(tf_i, node_fts, W_node, b_node.reshape(1, 3), gtn0, gtn1, gtn2,
      batch_vec.reshape(N_NODES, 1))


def _c1b_body(m0p_ref, m2p_ref, m0f_ref, m2f_ref):
    m0f_ref[...] = jnp.max(m0p_ref[...], axis=0, keepdims=True)
    m2f_ref[...] = jnp.max(m2p_ref[...], axis=0, keepdims=True)


def _c1b(m0p, m2p):
    return pl.pallas_call(
        _c1b_body,
        grid=(1,),
        in_specs=[
            pl.BlockSpec((NW, S0P), lambda i: (0, 0)),
            pl.BlockSpec((NW, G), lambda i: (0, 0)),
        ],
        out_specs=[
            pl.BlockSpec((1, S0P), lambda i: (0, 0)),
            pl.BlockSpec((1, G), lambda i: (0, 0)),
        ],
        out_shape=[
            jax.ShapeDtypeStruct((1, S0P), jnp.float32),
            jax.ShapeDtypeStruct((1, G), jnp.float32),
        ],
    )(m0p, m2p)


def _k3_denoms_states(e_idx, l0, l1, g0, g1, batch_vec, m0f, m2f, tf16):
    @functools.partial(
        pl.kernel,
        out_type=(
            jax.ShapeDtypeStruct((N_EDGES,), jnp.float32),
            jax.ShapeDtypeStruct((N_EDGES,), jnp.float32),
            jax.ShapeDtypeStruct((NW, S0P), jnp.float32),
            jax.ShapeDtypeStruct((NW, G), jnp.float32),
            jax.ShapeDtypeStruct((NW, S0P), jnp.float32),
            jax.ShapeDtypeStruct((NW, G), jnp.float32),
            jax.ShapeDtypeStruct((NW, L), jnp.float32),
            jax.ShapeDtypeStruct((NW, L), jnp.float32),
        ),
        mesh=_sc_mesh(),
        compiler_params=_SC_PARAMS,
        scratch_types=[
            pltpu.VMEM((CH,), jnp.int32),
            pltpu.VMEM((CH,), jnp.float32),
            pltpu.VMEM((CH,), jnp.float32),
            pltpu.VMEM((CH,), jnp.float32),
            pltpu.VMEM((CH,), jnp.float32),
            pltpu.VMEM((N_NODES,), jnp.int32),
            pltpu.VMEM((S0P,), jnp.float32),
            pltpu.VMEM((G,), jnp.float32),
            pltpu.VMEM((CH,), jnp.float32),
            pltpu.VMEM((CH,), jnp.float32),
            pltpu.VMEM((S0P,), jnp.float32),
            pltpu.VMEM((G,), jnp.float32),
            pltpu.VMEM((S0P,), jnp.float32),
            pltpu.VMEM((G,), jnp.float32),
            pltpu.VMEM((L,), jnp.int32),
            pltpu.VMEM((L,), jnp.float32),
        ],
    )
    def k(idx_h, l0_h, l1_h, g0_h, g1_h, bv_h, m0f_h, m2f_h, tf_h,
          s0_h, s1_h, d0p_h, d2p_h, g0p_h, g2p_h, dp0_h, dp1_h,
          idx_v, l0_v, l1_v, g0_v, g1_v, bv_v, m0f_v, m2f_v, s0_v, s1_v,
          d0a, d2a, g0a, g2a, tf_v, dt_v):
        wid = lax.axis_index("s") * 2 + lax.axis_index("c")
        base = wid * CH
        pltpu.sync_copy(idx_h.at[pl.ds(base, CH)], idx_v)
        pltpu.sync_copy(l0_h.at[pl.ds(base, CH)], l0_v)
        pltpu.sync_copy(l1_h.at[pl.ds(base, CH)], l1_v)
        pltpu.sync_copy(g0_h.at[pl.ds(base, CH)], g0_v)
        pltpu.sync_copy(g1_h.at[pl.ds(base, CH)], g1_v)
        pltpu.sync_copy(bv_h, bv_v)
        pltpu.sync_copy(m0f_h, m0f_v)
        pltpu.sync_copy(m2f_h, m2f_v)
        pltpu.sync_copy(tf_h, tf_v)
        _vfill(d0a, S0P, 0.0, jnp.float32)
        _vfill(d2a, G, 0.0, jnp.float32)
        _vfill(g0a, S0P, 0.0, jnp.float32)
        _vfill(g2a, G, 0.0, jnp.float32)
        tfv = tf_v[pl.ds(0, L)] != 0

        def step(j, carry):
            dv0, dv1 = carry
            sl = pl.ds(j * L, L)
            idx = idx_v[sl]
            v0 = l0_v[sl]
            gv0 = g0_v[sl]
            m0g = plsc.load_gather(m0f_v, [idx])
            s0_v[sl] = jnp.where(tfv, gv0, (v0 >= m0g).astype(jnp.float32))
            plsc.addupdate_scatter(d0a, [idx], jnp.exp(v0 - m0g))
            plsc.addupdate_scatter(g0a, [idx], gv0)
            idx2 = plsc.load_gather(bv_v, [idx])
            v1 = l1_v[sl]
            gv1 = g1_v[sl]
            m2g = plsc.load_gather(m2f_v, [idx2])
            s1_v[sl] = jnp.where(tfv, gv1, (v1 >= m2g).astype(jnp.float32))
            plsc.addupdate_scatter(d2a, [idx2], jnp.exp(v1 - m2g))
            plsc.addupdate_scatter(g2a, [idx2], gv1)
            return (dv0 + gv0 * v0, dv1 + gv1 * v1)

        zero = jnp.zeros((L,), jnp.float32)
        dv0, dv1 = lax.fori_loop(0, CH // L, step, (zero, zero))
        pltpu.sync_copy(s0_v, s0_h.at[pl.ds(base, CH)])
        pltpu.sync_copy(s1_v, s1_h.at[pl.ds(base, CH)])
        pltpu.sync_copy(d0a, d0p_h.at[wid])
        pltpu.sync_copy(d2a, d2p_h.at[wid])
        pltpu.sync_copy(g0a, g0p_h.at[wid])
        pltpu.sync_copy(g2a, g2p_h.at[wid])
        dt_v[pl.ds(0, L)] = dv0
        pltpu.sync_copy(dt_v, dp0_h.at[wid])
        dt_v[pl.ds(0, L)] = dv1
        pltpu.sync_copy(dt_v, dp1_h.at[wid])

    return k(e_idx, l0, l1, g0, g1, batch_vec, m0f, m2f, tf16)


def _c2_body(d0p_ref, d2p_ref, m0f_ref, g0p_ref, m2f_ref, g2p_ref,
             dp0_ref, dp1_ref, np_ref, out_ref):
    d0f = jnp.sum(d0p_ref[...], axis=0, keepdims=True)
    d2f = jnp.sum(d2p_ref[...], axis=0, keepdims=True)
    g0f = jnp.sum(g0p_ref[...], axis=0, keepdims=True)
    g2f = jnp.sum(g2p_ref[...], axis=0, keepdims=True)
    dot0 = jnp.sum(dp0_ref[...])
    dot1 = jnp.sum(dp1_ref[...])
    npv = np_ref[...]
    ii = lax.broadcasted_iota(jnp.int32, (1, G), 1)
    loss_node = jnp.sum(jnp.where(ii == 0, npv, 0.0))
    n0count = jnp.sum(jnp.where(ii == 1, npv, 0.0))
    m0f = m0f_ref[...]
    m2f = m2f_ref[...]
    loss_a = (-dot0 + jnp.sum(g0f * m0f) +
              jnp.sum(g0f * jnp.log(d0f + 1e-20))) / N_NODES
    loss_b = n0count * (-dot1 + jnp.sum(g2f * m2f) +
                        jnp.sum(g2f * jnp.log(d2f + 1e-20))) / G
    out_ref[...] = jnp.full((1, 1), loss_node + loss_a + loss_b, jnp.float32)


def _c2(d0p, d2p, m0f, g0p, m2f, g2p, dp0, dp1, npart):
    return pl.pallas_call(
        _c2_body,
        grid=(1,),
        in_specs=[
            pl.BlockSpec((NW, S0P), lambda i: (0, 0)),
            pl.BlockSpec((NW, G), lambda i: (0, 0)),
            pl.BlockSpec((1, S0P), lambda i: (0, 0)),
            pl.BlockSpec((NW, S0P), lambda i: (0, 0)),
            pl.BlockSpec((1, G), lambda i: (0, 0)),
            pl.BlockSpec((NW, G), lambda i: (0, 0)),
            pl.BlockSpec((NW, L), lambda i: (0, 0)),
            pl.BlockSpec((NW, L), lambda i: (0, 0)),
            pl.BlockSpec((1, G), lambda i: (0, 0)),
        ],
        out_specs=pl.BlockSpec((1, 1), lambda i: (0, 0)),
        out_shape=jax.ShapeDtypeStruct((1, 1), jnp.float32),
    )(d0p, d2p, m0f, g0p, m2f, g2p, dp0, dp1, npart)


def kernel(node_fts, edge_fts, node_hints, edge_hints, W_node, b_node, W_edge,
           b_edge, batch_vec, edge_index, processor_step, training_step,
           teacher_force):
    step = jnp.asarray(processor_step, jnp.int32)
    tf_i = jnp.asarray(teacher_force, jnp.int32).reshape(1)
    tf16 = jnp.broadcast_to(tf_i, (L,))
    batch_vec = batch_vec.astype(jnp.int32)

    g0 = lax.dynamic_slice(edge_hints, (0, step, 0),
                           (N_EDGES, 1, 1)).reshape(N_EDGES)
    g1 = lax.dynamic_slice(edge_hints, (0, step, 1),
                           (N_EDGES, 1, 1)).reshape(N_EDGES)
    gtn = [lax.dynamic_slice(node_hints, (0, step, k),
                             (N_NODES, 1, 1)).reshape(N_NODES, 1)
           for k in range(3)]

    l0, l1, e_idx = _edge_logits(edge_fts, W_edge, b_edge,
                                 edge_index.astype(jnp.int32))

    m0p, m2p = _k1_partials(e_idx, l0, l1, batch_vec)
    states_n, npart = _c1a(node_fts, W_node, b_node, gtn[0], gtn[1], gtn[2],
                           batch_vec, tf_i)
    m0f, m2f = _c1b(m0p, m2p)
    s0, s1, d0p, d2p, g0p, g2p, dp0, dp1 = _k3_denoms_states(
        e_idx, l0, l1, g0, g1, batch_vec, m0f.reshape(S0P), m2f.reshape(G),
        tf16)
    loss11 = _c2(d0p, d2p, m0f, g0p, m2f, g2p, dp0, dp1, npart)

    loss = loss11[0, 0]
    states_e = jnp.stack([s0, s1], axis=-1)
    return (states_n, states_e, loss)
